# baseline (speedup 1.0000x reference)
import functools
import numpy as np
import jax
import jax.numpy as jnp
from jax import lax
from jax.experimental import pallas as pl
from jax.experimental.pallas import tpu as pltpu
from jax.experimental.pallas import tpu_sc as plsc

_N = 10000
_E = 160000
_C = 32
_A = 8
_O = 32
_Q = 16
_K = 16
_NB = 8
_EA = 16
_MAX_R = 6.0
_SILU_NORM = 1.6768
_EMB_C = 1.14136 * float(np.exp(2.0))
_SQRT_NB = float(np.sqrt(_NB))
_INV_S24 = 1.0 / float(np.sqrt(_NB + _EA))
_INV_S128 = 1.0 / float(np.sqrt(128.0))
_INV_S32 = 1.0 / float(np.sqrt(_C))
_QD_SCALE = 1.0 / (_C * float(np.sqrt(_Q * _K)))
_SI_SCALE = 1.0 / float(np.sqrt(_C * _A))

_BN = 1000
_BE = 1000

_NW = 32
_EPW = _E // _NW
_GCH = 1000
_NCH = _EPW // _GCH
_NPT = _N // 16


def _sus(x):
    safe = jnp.where(x > 0.0, x, 1.0)
    return jnp.where(x > 0.0, jnp.exp(-1.0 / safe), 0.0)


def _silu(x):
    return x / (1.0 + jnp.exp(-x))


def _node_prep_body(x_ref, na_ref, pos_ref, wq_ref, wdot_ref, wsi_ref,
                    t_ref, si_ref):
    x = x_ref[...]
    na = na_ref[...]
    wqd = jnp.dot(wq_ref[...], wdot_ref[...], preferred_element_type=jnp.float32)
    qd = jnp.dot(x, wqd, preferred_element_type=jnp.float32) * _QD_SCALE
    t_ref[...] = jnp.concatenate(
        [x, qd, pos_ref[...], jnp.zeros((_BN, 128 - _C - _K - 3), jnp.float32)],
        axis=1)
    xa = jnp.concatenate([x * na[:, a:a + 1] for a in range(_A)], axis=1)
    si_ref[...] = jnp.dot(xa, wsi_ref[...], preferred_element_type=jnp.float32) * _SI_SCALE


def _node_prep(x, node_attr, pos, W_q, W_dot, wsi2):
    grid = (_N // _BN,)
    return pl.pallas_call(
        _node_prep_body,
        grid=grid,
        in_specs=[
            pl.BlockSpec((_BN, _C), lambda i: (i, 0)),
            pl.BlockSpec((_BN, _A), lambda i: (i, 0)),
            pl.BlockSpec((_BN, 3), lambda i: (i, 0)),
            pl.BlockSpec((_C, _Q), lambda i: (0, 0)),
            pl.BlockSpec((_Q, _K), lambda i: (0, 0)),
            pl.BlockSpec((_A * _C, _O), lambda i: (0, 0)),
        ],
        out_specs=[
            pl.BlockSpec((_BN, 128), lambda i: (i, 0)),
            pl.BlockSpec((_BN, _O), lambda i: (i, 0)),
        ],
        out_shape=[
            jax.ShapeDtypeStruct((_N, 128), jnp.float32),
            jax.ShapeDtypeStruct((_N, _O), jnp.float32),
        ],
    )(x, node_attr, pos, W_q, W_dot, wsi2)


_GC2 = 400
_GT = _EPW - (_EPW // _GC2) * _GC2


@functools.cache
def _build_sc_gather():
  mesh = plsc.VectorSubcoreMesh(core_axis_name="c", subcore_axis_name="s")

  @functools.partial(
    pl.kernel,
    mesh=mesh,
    out_type=(
        jax.ShapeDtypeStruct((_E, 128), jnp.float32),
        jax.ShapeDtypeStruct((_E, 128), jnp.float32),
    ),
    scratch_types=[
        pltpu.VMEM((_EPW,), jnp.int32),
        pltpu.VMEM((_EPW,), jnp.int32),
        pltpu.VMEM((_GC2, 128), jnp.float32),
        pltpu.VMEM((_GC2, 128), jnp.float32),
        pltpu.SemaphoreType.DMA,
        pltpu.SemaphoreType.DMA,
    ],
    compiler_params=pltpu.CompilerParams(needs_layout_passes=False),
  )
  def _sc_gather(src_h, dst_h, tt, gs_out, gd_out, src_v, dst_v, sbuf, dbuf,
                 sem1, sem2):
    wid = lax.axis_index("s") * 2 + lax.axis_index("c")
    base = wid * _EPW
    pltpu.sync_copy(src_h.at[pl.ds(base, _EPW)], src_v)
    pltpu.sync_copy(dst_h.at[pl.ds(base, _EPW)], dst_v)

    def chunk_body(ci, carry):
        off = ci * _GC2
        cp1 = pltpu.async_copy(tt.at[src_v.at[pl.ds(off, _GC2)]], sbuf, sem1)
        cp2 = pltpu.async_copy(tt.at[dst_v.at[pl.ds(off, _GC2)]], dbuf, sem2)
        cp1.wait()
        pltpu.sync_copy(sbuf, gs_out.at[pl.ds(base + off, _GC2)])
        cp2.wait()
        pltpu.sync_copy(dbuf, gd_out.at[pl.ds(base + off, _GC2)])
        return carry

    lax.fori_loop(0, _EPW // _GC2, chunk_body, 0)
    toff = (_EPW // _GC2) * _GC2
    cp1 = pltpu.async_copy(tt.at[src_v.at[pl.ds(toff, _GT)]],
                           sbuf.at[pl.ds(0, _GT)], sem1)
    cp2 = pltpu.async_copy(tt.at[dst_v.at[pl.ds(toff, _GT)]],
                           dbuf.at[pl.ds(0, _GT)], sem2)
    cp1.wait()
    pltpu.sync_copy(sbuf.at[pl.ds(0, _GT)], gs_out.at[pl.ds(base + toff, _GT)])
    cp2.wait()
    pltpu.sync_copy(dbuf.at[pl.ds(0, _GT)], gd_out.at[pl.ds(base + toff, _GT)])

  return _sc_gather


def _edge_body(gs_ref, gd_ref, ea_ref, w1k_ref, w2k_ref, w1v_ref,
               w2v_ref, out_ref):
    f32 = jnp.float32
    gs = gs_ref[...]
    gd = gd_ref[...]
    xs = gs[:, 0:_C]
    qd = gd[:, _C:_C + _K]
    dp = gs[:, _C + _K:_C + _K + 3] - gd[:, _C + _K:_C + _K + 3]
    l2 = jnp.sum(dp * dp, axis=1, keepdims=True) + 1e-24
    elen = jnp.sqrt(l2)

    jcol = lax.broadcasted_iota(jnp.int32, (_BE, _NB), 1).astype(f32)
    vals = (jcol + 1.0) * (_MAX_R / (_NB + 1))
    diff = (elen - vals) * ((_NB + 1) / _MAX_R)
    emb = (_EMB_C * _SQRT_NB) * _sus(diff + 1.0) * _sus(1.0 - diff)
    ed = jnp.concatenate([emb, ea_ref[...]], axis=1)

    hk = _silu(jnp.dot(ed, w1k_ref[...], preferred_element_type=f32) * _INV_S24) * _SILU_NORM
    wk2 = jnp.dot(hk, w2k_ref[...], preferred_element_type=f32) * _INV_S128
    hv = _silu(jnp.dot(ed, w1v_ref[...], preferred_element_type=f32) * _INV_S24) * _SILU_NORM
    wv2 = jnp.dot(hv, w2v_ref[...], preferred_element_type=f32) * _INV_S128

    rk_r = lax.broadcasted_iota(jnp.int32, (_C, _C * _K), 0)
    rk_c = lax.broadcasted_iota(jnp.int32, (_C, _C * _K), 1)
    Rk = (rk_c // _K == rk_r).astype(f32)
    sk_r = lax.broadcasted_iota(jnp.int32, (_C * _K, _K), 0)
    sk_c = lax.broadcasted_iota(jnp.int32, (_C * _K, _K), 1)
    Sk = (sk_r % _K == sk_c).astype(f32)
    xs_k = jnp.dot(xs, Rk, preferred_element_type=f32)
    kraw = jnp.dot(wk2 * xs_k, Sk, preferred_element_type=f32)

    rv_r = lax.broadcasted_iota(jnp.int32, (_C, _C * _O), 0)
    rv_c = lax.broadcasted_iota(jnp.int32, (_C, _C * _O), 1)
    Rv = (rv_c // _O == rv_r).astype(f32)
    sv_r = lax.broadcasted_iota(jnp.int32, (_C * _O, _O), 0)
    sv_c = lax.broadcasted_iota(jnp.int32, (_C * _O, _O), 1)
    Sv = (sv_r % _O == sv_c).astype(f32)
    xs_v = jnp.dot(xs, Rv, preferred_element_type=f32)
    vraw = jnp.dot(wv2 * xs_v, Sv, preferred_element_type=f32)

    temp = jnp.sum(qd * kraw, axis=1, keepdims=True)
    ewc = _sus(10.0 * (1.0 - elen * (1.0 / _MAX_R)))
    t2 = ewc * temp
    expv = jnp.exp(t2)
    sexp = jnp.exp(0.5 * t2)
    num = sexp * vraw * _INV_S32
    out_ref[...] = jnp.concatenate(
        [num, expv, jnp.zeros((_BE, 15), f32)], axis=1)


def _edge_pass(gs, gd, edge_attr, W1_k, W2_k, W1_v, W2_v):
    grid = (_E // _BE,)
    return pl.pallas_call(
        _edge_body,
        grid=grid,
        in_specs=[
            pl.BlockSpec((_BE, 128), lambda i: (i, 0)),
            pl.BlockSpec((_BE, 128), lambda i: (i, 0)),
            pl.BlockSpec((_BE, _EA), lambda i: (i, 0)),
            pl.BlockSpec((_NB + _EA, 128), lambda i: (0, 0)),
            pl.BlockSpec((128, _C * _K), lambda i: (0, 0)),
            pl.BlockSpec((_NB + _EA, 128), lambda i: (0, 0)),
            pl.BlockSpec((128, _C * _O), lambda i: (0, 0)),
        ],
        out_specs=pl.BlockSpec((_BE, 48), lambda i: (i, 0)),
        out_shape=jax.ShapeDtypeStruct((_E, 48), jnp.float32),
    )(gs, gd, edge_attr, W1_k, W2_k, W1_v, W2_v)


@functools.cache
def _build_sc_scatter():
  mesh = plsc.VectorSubcoreMesh(core_axis_name="c", subcore_axis_name="s")

  @functools.partial(
    pl.kernel,
    mesh=mesh,
    out_type=jax.ShapeDtypeStruct((2, _N, 48), jnp.float32),
    scratch_types=[
        pltpu.VMEM((_GCH,), jnp.int32),
        pltpu.VMEM((_GCH, 48), jnp.float32),
        pltpu.VMEM((_NPT, 48), jnp.float32),
        pltpu.VMEM_SHARED((_N, 48), jnp.float32),
        pltpu.SemaphoreType.DMA,
    ],
    compiler_params=pltpu.CompilerParams(needs_layout_passes=False, use_tc_tiling_on_sc=False),
  )
  def _sc_scatter(dst_h, rows_h, out_h, dstc, rowsv, zb, table, sem):
    cid = lax.axis_index("c")
    sid = lax.axis_index("s")
    wid = sid * 2 + cid
    base = wid * _EPW

    zero16 = jnp.zeros((16,), jnp.float32)

    def zb_body(i, carry):
        r = i // 3
        c = (i % 3) * 16
        zb[r, pl.ds(c, 16)] = zero16
        return carry

    lax.fori_loop(0, _NPT * 3, zb_body, 0)
    pltpu.sync_copy(zb, table.at[pl.ds(sid * _NPT, _NPT)])
    plsc.subcore_barrier()

    def chunk_body(ci, carry):
        off = base + ci * _GCH
        pltpu.sync_copy(dst_h.at[pl.ds(off, _GCH)], dstc)
        pltpu.sync_copy(rows_h.at[pl.ds(off, _GCH)], rowsv)
        pltpu.sync_copy(rowsv, table.at[dstc], add=True)
        return carry

    lax.fori_loop(0, _NCH, chunk_body, 0)
    plsc.subcore_barrier()
    pltpu.sync_copy(table.at[pl.ds(sid * _NPT, _NPT)],
                    out_h.at[cid, pl.ds(sid * _NPT, _NPT)])

  return _sc_scatter


def _combine_body(s0_ref, s1_ref, si_ref, out_ref):
    s = s0_ref[0] + s1_ref[0]
    z = s[:, 32:33]
    zz = jnp.where(z == 0.0, 1.0, z)
    out_ref[...] = si_ref[...] + s[:, 0:_O] * lax.rsqrt(zz)


def _combine(s48, si):
    grid = (_N // _BN,)
    return pl.pallas_call(
        _combine_body,
        grid=grid,
        in_specs=[
            pl.BlockSpec((1, _BN, 48), lambda i: (0, i, 0)),
            pl.BlockSpec((1, _BN, 48), lambda i: (1, i, 0)),
            pl.BlockSpec((_BN, _O), lambda i: (i, 0)),
        ],
        out_specs=pl.BlockSpec((_BN, _O), lambda i: (i, 0)),
        out_shape=jax.ShapeDtypeStruct((_N, _O), jnp.float32),
    )(s48, s48, si)


def kernel(x, pos, node_attr, edge_index, edge_attr, batch, W_q, W_si,
           W1_k, W2_k, W1_v, W2_v, W_dot):
    wsi2 = jnp.transpose(W_si, (1, 0, 2)).reshape(_A * _C, _O)
    src = edge_index[0]
    dst = edge_index[1]
    tt, si = _node_prep(x, node_attr, pos, W_q, W_dot, wsi2)
    gs, gd = _build_sc_gather()(src, dst, tt)
    out48 = _edge_pass(gs, gd, edge_attr, W1_k, W2_k, W1_v, W2_v)
    s48 = _build_sc_scatter()(dst, out48)
    return _combine(s48, si)


# speedup vs baseline: 1.1823x; 1.1823x over previous
import functools
import numpy as np
import jax
import jax.numpy as jnp
from jax import lax
from jax.experimental import pallas as pl
from jax.experimental.pallas import tpu as pltpu
from jax.experimental.pallas import tpu_sc as plsc

_N = 10000
_E = 160000
_C = 32
_A = 8
_O = 32
_Q = 16
_K = 16
_NB = 8
_EA = 16
_MAX_R = 6.0
_SILU_NORM = 1.6768
_EMB_C = 1.14136 * float(np.exp(2.0))
_SQRT_NB = float(np.sqrt(_NB))
_INV_S24 = 1.0 / float(np.sqrt(_NB + _EA))
_INV_S128 = 1.0 / float(np.sqrt(128.0))
_INV_S32 = 1.0 / float(np.sqrt(_C))
_QD_SCALE = 1.0 / (_C * float(np.sqrt(_Q * _K)))
_SI_SCALE = 1.0 / float(np.sqrt(_C * _A))

_BN = 1000
_BE = 1000

_NW = 32
_EPW = _E // _NW
_GCH = 1000
_NCH = _EPW // _GCH
_NPT = _N // 16


def _sus(x):
    safe = jnp.where(x > 0.0, x, 1.0)
    return jnp.where(x > 0.0, jnp.exp(-1.0 / safe), 0.0)


def _silu(x):
    return x / (1.0 + jnp.exp(-x))


def _node_prep_body(x_ref, na_ref, pos_ref, wq_ref, wdot_ref, wsi_ref,
                    t_ref, si_ref):
    x = x_ref[...]
    na = na_ref[...]
    wqd = jnp.dot(wq_ref[...], wdot_ref[...], preferred_element_type=jnp.float32)
    qd = jnp.dot(x, wqd, preferred_element_type=jnp.float32) * _QD_SCALE
    t_ref[...] = jnp.concatenate(
        [x, qd, pos_ref[...], jnp.zeros((_BN, 128 - _C - _K - 3), jnp.float32)],
        axis=1)
    xa = jnp.concatenate([x * na[:, a:a + 1] for a in range(_A)], axis=1)
    si_ref[...] = jnp.dot(xa, wsi_ref[...], preferred_element_type=jnp.float32) * _SI_SCALE


def _node_prep(x, node_attr, pos, W_q, W_dot, wsi2):
    grid = (_N // _BN,)
    return pl.pallas_call(
        _node_prep_body,
        grid=grid,
        in_specs=[
            pl.BlockSpec((_BN, _C), lambda i: (i, 0)),
            pl.BlockSpec((_BN, _A), lambda i: (i, 0)),
            pl.BlockSpec((_BN, 3), lambda i: (i, 0)),
            pl.BlockSpec((_C, _Q), lambda i: (0, 0)),
            pl.BlockSpec((_Q, _K), lambda i: (0, 0)),
            pl.BlockSpec((_A * _C, _O), lambda i: (0, 0)),
        ],
        out_specs=[
            pl.BlockSpec((_BN, 128), lambda i: (i, 0)),
            pl.BlockSpec((_BN, _O), lambda i: (i, 0)),
        ],
        out_shape=[
            jax.ShapeDtypeStruct((_N, 128), jnp.float32),
            jax.ShapeDtypeStruct((_N, _O), jnp.float32),
        ],
    )(x, node_attr, pos, W_q, W_dot, wsi2)


_GC2 = 400
_GT = _EPW - (_EPW // _GC2) * _GC2


@functools.cache
def _build_sc_gather():
  mesh = plsc.VectorSubcoreMesh(core_axis_name="c", subcore_axis_name="s")

  @functools.partial(
    pl.kernel,
    mesh=mesh,
    out_type=(
        jax.ShapeDtypeStruct((_E, 128), jnp.float32),
        jax.ShapeDtypeStruct((_E, 128), jnp.float32),
    ),
    scratch_types=[
        pltpu.VMEM((_EPW,), jnp.int32),
        pltpu.VMEM((_EPW,), jnp.int32),
        pltpu.VMEM((_GC2, 128), jnp.float32),
        pltpu.VMEM((_GC2, 128), jnp.float32),
        pltpu.SemaphoreType.DMA,
        pltpu.SemaphoreType.DMA,
    ],
    compiler_params=pltpu.CompilerParams(needs_layout_passes=False),
  )
  def _sc_gather(src_h, dst_h, tt, gs_out, gd_out, src_v, dst_v, sbuf, dbuf,
                 sem1, sem2):
    wid = lax.axis_index("s") * 2 + lax.axis_index("c")
    base = wid * _EPW
    pltpu.sync_copy(src_h.at[pl.ds(base, _EPW)], src_v)
    pltpu.sync_copy(dst_h.at[pl.ds(base, _EPW)], dst_v)

    def chunk_body(ci, carry):
        off = ci * _GC2
        cp1 = pltpu.async_copy(tt.at[src_v.at[pl.ds(off, _GC2)]], sbuf, sem1)
        cp2 = pltpu.async_copy(tt.at[dst_v.at[pl.ds(off, _GC2)]], dbuf, sem2)
        cp1.wait()
        pltpu.sync_copy(sbuf, gs_out.at[pl.ds(base + off, _GC2)])
        cp2.wait()
        pltpu.sync_copy(dbuf, gd_out.at[pl.ds(base + off, _GC2)])
        return carry

    lax.fori_loop(0, _EPW // _GC2, chunk_body, 0)
    toff = (_EPW // _GC2) * _GC2
    cp1 = pltpu.async_copy(tt.at[src_v.at[pl.ds(toff, _GT)]],
                           sbuf.at[pl.ds(0, _GT)], sem1)
    cp2 = pltpu.async_copy(tt.at[dst_v.at[pl.ds(toff, _GT)]],
                           dbuf.at[pl.ds(0, _GT)], sem2)
    cp1.wait()
    pltpu.sync_copy(sbuf.at[pl.ds(0, _GT)], gs_out.at[pl.ds(base + toff, _GT)])
    cp2.wait()
    pltpu.sync_copy(dbuf.at[pl.ds(0, _GT)], gd_out.at[pl.ds(base + toff, _GT)])

  return _sc_gather


def _edge_body(gs_ref, gd_ref, ea_ref, w1k_ref, w2k_ref, w1v_ref,
               w2v_ref, out_ref):
    f32 = jnp.float32
    bf16 = jnp.bfloat16
    gs = gs_ref[...]
    gd = gd_ref[...]

    px_r = lax.broadcasted_iota(jnp.int32, (128, _C), 0)
    px_c = lax.broadcasted_iota(jnp.int32, (128, _C), 1)
    Px = (px_r == px_c).astype(f32)
    pq_r = lax.broadcasted_iota(jnp.int32, (128, _K), 0)
    pq_c = lax.broadcasted_iota(jnp.int32, (128, _K), 1)
    Pq = (pq_r == pq_c + _C).astype(f32)
    pp_r = lax.broadcasted_iota(jnp.int32, (128, 1), 0)
    Pp = ((pp_r >= _C + _K) & (pp_r < _C + _K + 3)).astype(f32)

    xs = jnp.dot(gs, Px, preferred_element_type=f32)
    qd = jnp.dot(gd, Pq, preferred_element_type=f32)
    dp = gs - gd
    l2 = jnp.dot(dp * dp, Pp, preferred_element_type=f32) + 1e-24
    elen = jnp.sqrt(l2)

    jcol = lax.broadcasted_iota(jnp.int32, (_BE, _NB), 1).astype(f32)
    vals = (jcol + 1.0) * (_MAX_R / (_NB + 1))
    diff = (elen - vals) * ((_NB + 1) / _MAX_R)
    emb = (_EMB_C * _SQRT_NB) * _sus(diff + 1.0) * _sus(1.0 - diff)
    ed = jnp.concatenate([emb, ea_ref[...]], axis=1)

    hk = _silu(jnp.dot(ed, w1k_ref[...], preferred_element_type=f32) * _INV_S24) * _SILU_NORM
    wk2 = jnp.dot(hk, w2k_ref[...], preferred_element_type=f32) * _INV_S128
    hv = _silu(jnp.dot(ed, w1v_ref[...], preferred_element_type=f32) * _INV_S24) * _SILU_NORM
    wv2 = jnp.dot(hv.astype(bf16), w2v_ref[...].astype(bf16),
                  preferred_element_type=f32) * _INV_S128

    rk_r = lax.broadcasted_iota(jnp.int32, (_C, _C * _K), 0)
    rk_c = lax.broadcasted_iota(jnp.int32, (_C, _C * _K), 1)
    Rk = (rk_c // _K == rk_r).astype(f32)
    sk_r = lax.broadcasted_iota(jnp.int32, (_C * _K, _K), 0)
    sk_c = lax.broadcasted_iota(jnp.int32, (_C * _K, _K), 1)
    Sk = (sk_r % _K == sk_c).astype(f32)
    xs_k = jnp.dot(xs, Rk, preferred_element_type=f32)
    kraw = jnp.dot(wk2 * xs_k, Sk, preferred_element_type=f32)

    rv_r = lax.broadcasted_iota(jnp.int32, (_C, _C * _O), 0)
    rv_c = lax.broadcasted_iota(jnp.int32, (_C, _C * _O), 1)
    Rv = (rv_c // _O == rv_r).astype(bf16)
    sv_r = lax.broadcasted_iota(jnp.int32, (_C * _O, _O), 0)
    sv_c = lax.broadcasted_iota(jnp.int32, (_C * _O, _O), 1)
    Sv = (sv_r % _O == sv_c).astype(bf16)
    xs_v = jnp.dot(xs.astype(bf16), Rv, preferred_element_type=f32)
    vraw = jnp.dot((wv2 * xs_v).astype(bf16), Sv,
                   preferred_element_type=f32)

    temp = jnp.sum(qd * kraw, axis=1, keepdims=True)
    ewc = _sus(10.0 * (1.0 - elen * (1.0 / _MAX_R)))
    t2 = ewc * temp
    expv = jnp.exp(t2)
    sexp = jnp.exp(0.5 * t2)
    num = sexp * vraw * _INV_S32
    out_ref[...] = jnp.concatenate(
        [num, expv, jnp.zeros((_BE, 15), f32)], axis=1)


def _edge_pass(gs, gd, edge_attr, W1_k, W2_k, W1_v, W2_v):
    grid = (_E // _BE,)
    return pl.pallas_call(
        _edge_body,
        grid=grid,
        in_specs=[
            pl.BlockSpec((_BE, 128), lambda i: (i, 0)),
            pl.BlockSpec((_BE, 128), lambda i: (i, 0)),
            pl.BlockSpec((_BE, _EA), lambda i: (i, 0)),
            pl.BlockSpec((_NB + _EA, 128), lambda i: (0, 0)),
            pl.BlockSpec((128, _C * _K), lambda i: (0, 0)),
            pl.BlockSpec((_NB + _EA, 128), lambda i: (0, 0)),
            pl.BlockSpec((128, _C * _O), lambda i: (0, 0)),
        ],
        out_specs=pl.BlockSpec((_BE, 48), lambda i: (i, 0)),
        out_shape=jax.ShapeDtypeStruct((_E, 48), jnp.float32),
    )(gs, gd, edge_attr, W1_k, W2_k, W1_v, W2_v)


@functools.cache
def _build_sc_scatter():
  mesh = plsc.VectorSubcoreMesh(core_axis_name="c", subcore_axis_name="s")

  @functools.partial(
    pl.kernel,
    mesh=mesh,
    out_type=jax.ShapeDtypeStruct((2, _N, 48), jnp.float32),
    scratch_types=[
        pltpu.VMEM((_GCH,), jnp.int32),
        pltpu.VMEM((_GCH, 48), jnp.float32),
        pltpu.VMEM((_NPT, 48), jnp.float32),
        pltpu.VMEM_SHARED((_N, 48), jnp.float32),
        pltpu.SemaphoreType.DMA,
    ],
    compiler_params=pltpu.CompilerParams(needs_layout_passes=False, use_tc_tiling_on_sc=False),
  )
  def _sc_scatter(dst_h, rows_h, out_h, dstc, rowsv, zb, table, sem):
    cid = lax.axis_index("c")
    sid = lax.axis_index("s")
    wid = sid * 2 + cid
    base = wid * _EPW

    zero16 = jnp.zeros((16,), jnp.float32)

    def zb_body(i, carry):
        r = i // 3
        c = (i % 3) * 16
        zb[r, pl.ds(c, 16)] = zero16
        return carry

    lax.fori_loop(0, _NPT * 3, zb_body, 0)
    pltpu.sync_copy(zb, table.at[pl.ds(sid * _NPT, _NPT)])
    plsc.subcore_barrier()

    def chunk_body(ci, carry):
        off = base + ci * _GCH
        pltpu.sync_copy(dst_h.at[pl.ds(off, _GCH)], dstc)
        pltpu.sync_copy(rows_h.at[pl.ds(off, _GCH)], rowsv)
        pltpu.sync_copy(rowsv, table.at[dstc], add=True)
        return carry

    lax.fori_loop(0, _NCH, chunk_body, 0)
    plsc.subcore_barrier()
    pltpu.sync_copy(table.at[pl.ds(sid * _NPT, _NPT)],
                    out_h.at[cid, pl.ds(sid * _NPT, _NPT)])

  return _sc_scatter


def _combine_body(s0_ref, s1_ref, si_ref, out_ref):
    s = s0_ref[0] + s1_ref[0]
    z = s[:, 32:33]
    zz = jnp.where(z == 0.0, 1.0, z)
    out_ref[...] = si_ref[...] + s[:, 0:_O] * lax.rsqrt(zz)


def _combine(s48, si):
    grid = (_N // _BN,)
    return pl.pallas_call(
        _combine_body,
        grid=grid,
        in_specs=[
            pl.BlockSpec((1, _BN, 48), lambda i: (0, i, 0)),
            pl.BlockSpec((1, _BN, 48), lambda i: (1, i, 0)),
            pl.BlockSpec((_BN, _O), lambda i: (i, 0)),
        ],
        out_specs=pl.BlockSpec((_BN, _O), lambda i: (i, 0)),
        out_shape=jax.ShapeDtypeStruct((_N, _O), jnp.float32),
    )(s48, s48, si)


def kernel(x, pos, node_attr, edge_index, edge_attr, batch, W_q, W_si,
           W1_k, W2_k, W1_v, W2_v, W_dot):
    wsi2 = jnp.transpose(W_si, (1, 0, 2)).reshape(_A * _C, _O)
    src = edge_index[0]
    dst = edge_index[1]
    tt, si = _node_prep(x, node_attr, pos, W_q, W_dot, wsi2)
    gs, gd = _build_sc_gather()(src, dst, tt)
    out48 = _edge_pass(gs, gd, edge_attr, W1_k, W2_k, W1_v, W2_v)
    s48 = _build_sc_scatter()(dst, out48)
    return _combine(s48, si)


# speedup vs baseline: 1.2836x; 1.0857x over previous
import functools
import numpy as np
import jax
import jax.numpy as jnp
from jax import lax
from jax.experimental import pallas as pl
from jax.experimental.pallas import tpu as pltpu
from jax.experimental.pallas import tpu_sc as plsc

_N = 10000
_E = 160000
_C = 32
_A = 8
_O = 32
_Q = 16
_K = 16
_NB = 8
_EA = 16
_MAX_R = 6.0
_SILU_NORM = 1.6768
_EMB_C = 1.14136 * float(np.exp(2.0))
_SQRT_NB = float(np.sqrt(_NB))
_INV_S24 = 1.0 / float(np.sqrt(_NB + _EA))
_INV_S128 = 1.0 / float(np.sqrt(128.0))
_INV_S32 = 1.0 / float(np.sqrt(_C))
_QD_SCALE = 1.0 / (_C * float(np.sqrt(_Q * _K)))
_SI_SCALE = 1.0 / float(np.sqrt(_C * _A))

_BN = 1000
_BE = 1280

_NW = 32
_EPW = _E // _NW
_GCH = 1000
_NCH = _EPW // _GCH
_NPT = _N // 16


def _sus(x):
    safe = jnp.where(x > 0.0, x, 1.0)
    return jnp.where(x > 0.0, jnp.exp(-1.0 / safe), 0.0)


def _silu(x):
    return x / (1.0 + jnp.exp(-x))


def _node_prep_body(x_ref, na_ref, pos_ref, wq_ref, wdot_ref, wsi_ref,
                    t_ref, si_ref):
    x = x_ref[...]
    na = na_ref[...]
    wqd = jnp.dot(wq_ref[...], wdot_ref[...], preferred_element_type=jnp.float32)
    qd = jnp.dot(x, wqd, preferred_element_type=jnp.float32) * _QD_SCALE
    t_ref[...] = jnp.concatenate(
        [x, qd, pos_ref[...], jnp.zeros((_BN, 128 - _C - _K - 3), jnp.float32)],
        axis=1)
    xa = jnp.concatenate([x * na[:, a:a + 1] for a in range(_A)], axis=1)
    si_ref[...] = jnp.dot(xa, wsi_ref[...], preferred_element_type=jnp.float32) * _SI_SCALE


def _node_prep(x, node_attr, pos, W_q, W_dot, wsi2):
    grid = (_N // _BN,)
    return pl.pallas_call(
        _node_prep_body,
        grid=grid,
        in_specs=[
            pl.BlockSpec((_BN, _C), lambda i: (i, 0)),
            pl.BlockSpec((_BN, _A), lambda i: (i, 0)),
            pl.BlockSpec((_BN, 3), lambda i: (i, 0)),
            pl.BlockSpec((_C, _Q), lambda i: (0, 0)),
            pl.BlockSpec((_Q, _K), lambda i: (0, 0)),
            pl.BlockSpec((_A * _C, _O), lambda i: (0, 0)),
        ],
        out_specs=[
            pl.BlockSpec((_BN, 128), lambda i: (i, 0)),
            pl.BlockSpec((_BN, _O), lambda i: (i, 0)),
        ],
        out_shape=[
            jax.ShapeDtypeStruct((_N, 128), jnp.float32),
            jax.ShapeDtypeStruct((_N, _O), jnp.float32),
        ],
    )(x, node_attr, pos, W_q, W_dot, wsi2)


_GC2 = 400
_GT = _EPW - (_EPW // _GC2) * _GC2


@functools.cache
def _build_sc_gather():
  mesh = plsc.VectorSubcoreMesh(core_axis_name="c", subcore_axis_name="s")

  @functools.partial(
    pl.kernel,
    mesh=mesh,
    out_type=(
        jax.ShapeDtypeStruct((_E, 128), jnp.float32),
        jax.ShapeDtypeStruct((_E, 128), jnp.float32),
    ),
    scratch_types=[
        pltpu.VMEM((_EPW,), jnp.int32),
        pltpu.VMEM((_EPW,), jnp.int32),
        pltpu.VMEM((_GC2, 128), jnp.float32),
        pltpu.VMEM((_GC2, 128), jnp.float32),
        pltpu.SemaphoreType.DMA,
        pltpu.SemaphoreType.DMA,
    ],
    compiler_params=pltpu.CompilerParams(needs_layout_passes=False),
  )
  def _sc_gather(src_h, dst_h, tt, gs_out, gd_out, src_v, dst_v, sbuf, dbuf,
                 sem1, sem2):
    wid = lax.axis_index("s") * 2 + lax.axis_index("c")
    base = wid * _EPW
    pltpu.sync_copy(src_h.at[pl.ds(base, _EPW)], src_v)
    pltpu.sync_copy(dst_h.at[pl.ds(base, _EPW)], dst_v)

    def chunk_body(ci, carry):
        off = ci * _GC2
        cp1 = pltpu.async_copy(tt.at[src_v.at[pl.ds(off, _GC2)]], sbuf, sem1)
        cp2 = pltpu.async_copy(tt.at[dst_v.at[pl.ds(off, _GC2)]], dbuf, sem2)
        cp1.wait()
        pltpu.sync_copy(sbuf, gs_out.at[pl.ds(base + off, _GC2)])
        cp2.wait()
        pltpu.sync_copy(dbuf, gd_out.at[pl.ds(base + off, _GC2)])
        return carry

    lax.fori_loop(0, _EPW // _GC2, chunk_body, 0)
    toff = (_EPW // _GC2) * _GC2
    cp1 = pltpu.async_copy(tt.at[src_v.at[pl.ds(toff, _GT)]],
                           sbuf.at[pl.ds(0, _GT)], sem1)
    cp2 = pltpu.async_copy(tt.at[dst_v.at[pl.ds(toff, _GT)]],
                           dbuf.at[pl.ds(0, _GT)], sem2)
    cp1.wait()
    pltpu.sync_copy(sbuf.at[pl.ds(0, _GT)], gs_out.at[pl.ds(base + toff, _GT)])
    cp2.wait()
    pltpu.sync_copy(dbuf.at[pl.ds(0, _GT)], gd_out.at[pl.ds(base + toff, _GT)])

  return _sc_gather


def _edge_body(gs_ref, gd_ref, eat_ref, w1e_ref, w1a_ref, w2k_ref,
               w2v_ref, out_ref):
    f32 = jnp.float32
    bf16 = jnp.bfloat16
    gs = gs_ref[...]
    gd = gd_ref[...]

    px_r = lax.broadcasted_iota(jnp.int32, (128, _C), 0)
    px_c = lax.broadcasted_iota(jnp.int32, (128, _C), 1)
    Px = (px_r == px_c).astype(f32)
    pq_r = lax.broadcasted_iota(jnp.int32, (128, _K), 0)
    pq_c = lax.broadcasted_iota(jnp.int32, (128, _K), 1)
    Pq = (pq_r == pq_c + _C).astype(f32)
    pp_r = lax.broadcasted_iota(jnp.int32, (128, 1), 0)
    Pp = ((pp_r >= _C + _K) & (pp_r < _C + _K + 3)).astype(f32)

    xs = jnp.dot(gs, Px, preferred_element_type=f32)
    qd = jnp.dot(gd, Pq, preferred_element_type=f32)
    dp = gs - gd
    l2 = jnp.dot(dp * dp, Pp, preferred_element_type=f32) + 1e-24
    elen = jnp.sqrt(l2)

    jcol = lax.broadcasted_iota(jnp.int32, (_BE, _NB), 1).astype(f32)
    vals = (jcol + 1.0) * (_MAX_R / (_NB + 1))
    diff = (elen - vals) * ((_NB + 1) / _MAX_R)
    dd = 1.0 - diff * diff
    inside = dd > 0.0
    dd_safe = jnp.where(inside, dd, 1.0)
    emb = jnp.where(inside,
                    (_EMB_C * _SQRT_NB) * jnp.exp(-2.0 / dd_safe), 0.0)

    pre = (jnp.dot(emb, w1e_ref[...], preferred_element_type=f32)
           + lax.dot_general(eat_ref[...], w1a_ref[...],
                             (((0,), (0,)), ((), ())),
                             preferred_element_type=f32))
    h = _silu(pre * _INV_S24) * _SILU_NORM
    hk = h[:, 0:128]
    hv = h[:, 128:256]
    wk2 = jnp.dot(hk, w2k_ref[...], preferred_element_type=f32) * _INV_S128
    wv2 = jnp.dot(hv.astype(bf16), w2v_ref[...].astype(bf16),
                  preferred_element_type=f32) * _INV_S128

    rk_r = lax.broadcasted_iota(jnp.int32, (_C, _C * _K), 0)
    rk_c = lax.broadcasted_iota(jnp.int32, (_C, _C * _K), 1)
    Rk = (rk_c // _K == rk_r).astype(f32)
    sk_r = lax.broadcasted_iota(jnp.int32, (_C * _K, _K), 0)
    sk_c = lax.broadcasted_iota(jnp.int32, (_C * _K, _K), 1)
    Sk = (sk_r % _K == sk_c).astype(f32)
    xs_k = jnp.dot(xs, Rk, preferred_element_type=f32)
    kraw = jnp.dot(wk2 * xs_k, Sk, preferred_element_type=f32)

    rv_r = lax.broadcasted_iota(jnp.int32, (_C, _C * _O), 0)
    rv_c = lax.broadcasted_iota(jnp.int32, (_C, _C * _O), 1)
    Rv = (rv_c // _O == rv_r).astype(bf16)
    sv_r = lax.broadcasted_iota(jnp.int32, (_C * _O, _O), 0)
    sv_c = lax.broadcasted_iota(jnp.int32, (_C * _O, _O), 1)
    Sv = (sv_r % _O == sv_c).astype(bf16)
    xs_v = jnp.dot(xs.astype(bf16), Rv, preferred_element_type=f32)
    vraw = jnp.dot((wv2 * xs_v).astype(bf16), Sv,
                   preferred_element_type=f32)

    temp = jnp.sum(qd * kraw, axis=1, keepdims=True)
    ewc = _sus(10.0 * (1.0 - elen * (1.0 / _MAX_R)))
    t2 = ewc * temp
    expv = jnp.exp(t2)
    sexp = jnp.exp(0.5 * t2)
    num = sexp * vraw * _INV_S32
    out_ref[...] = jnp.concatenate(
        [num, expv, jnp.zeros((_BE, 15), f32)], axis=1)


def _edge_pass(gs, gd, ea_t, w1e, w1a, W2_k, W2_v):
    grid = (_E // _BE,)
    return pl.pallas_call(
        _edge_body,
        grid=grid,
        in_specs=[
            pl.BlockSpec((_BE, 128), lambda i: (i, 0)),
            pl.BlockSpec((_BE, 128), lambda i: (i, 0)),
            pl.BlockSpec((_EA, _BE), lambda i: (0, i)),
            pl.BlockSpec((_NB, 256), lambda i: (0, 0)),
            pl.BlockSpec((_EA, 256), lambda i: (0, 0)),
            pl.BlockSpec((128, _C * _K), lambda i: (0, 0)),
            pl.BlockSpec((128, _C * _O), lambda i: (0, 0)),
        ],
        out_specs=pl.BlockSpec((_BE, 48), lambda i: (i, 0)),
        out_shape=jax.ShapeDtypeStruct((_E, 48), jnp.float32),
    )(gs, gd, ea_t, w1e, w1a, W2_k, W2_v)


@functools.cache
def _build_sc_scatter():
  mesh = plsc.VectorSubcoreMesh(core_axis_name="c", subcore_axis_name="s")

  @functools.partial(
    pl.kernel,
    mesh=mesh,
    out_type=jax.ShapeDtypeStruct((2, _N, 48), jnp.float32),
    scratch_types=[
        pltpu.VMEM((_GCH,), jnp.int32),
        pltpu.VMEM((_GCH, 48), jnp.float32),
        pltpu.VMEM((_NPT, 48), jnp.float32),
        pltpu.VMEM_SHARED((_N, 48), jnp.float32),
        pltpu.SemaphoreType.DMA,
    ],
    compiler_params=pltpu.CompilerParams(needs_layout_passes=False, use_tc_tiling_on_sc=False),
  )
  def _sc_scatter(dst_h, rows_h, out_h, dstc, rowsv, zb, table, sem):
    cid = lax.axis_index("c")
    sid = lax.axis_index("s")
    wid = sid * 2 + cid
    base = wid * _EPW

    zero16 = jnp.zeros((16,), jnp.float32)

    def zb_body(i, carry):
        r = i // 3
        c = (i % 3) * 16
        zb[r, pl.ds(c, 16)] = zero16
        return carry

    lax.fori_loop(0, _NPT * 3, zb_body, 0)
    pltpu.sync_copy(zb, table.at[pl.ds(sid * _NPT, _NPT)])
    plsc.subcore_barrier()

    def chunk_body(ci, carry):
        off = base + ci * _GCH
        pltpu.sync_copy(dst_h.at[pl.ds(off, _GCH)], dstc)
        pltpu.sync_copy(rows_h.at[pl.ds(off, _GCH)], rowsv)
        pltpu.sync_copy(rowsv, table.at[dstc], add=True)
        return carry

    lax.fori_loop(0, _NCH, chunk_body, 0)
    plsc.subcore_barrier()
    pltpu.sync_copy(table.at[pl.ds(sid * _NPT, _NPT)],
                    out_h.at[cid, pl.ds(sid * _NPT, _NPT)])

  return _sc_scatter


def _combine_body(s0_ref, s1_ref, si_ref, out_ref):
    s = s0_ref[0] + s1_ref[0]
    z = s[:, 32:33]
    zz = jnp.where(z == 0.0, 1.0, z)
    out_ref[...] = si_ref[...] + s[:, 0:_O] * lax.rsqrt(zz)


def _combine(s48, si):
    grid = (_N // _BN,)
    return pl.pallas_call(
        _combine_body,
        grid=grid,
        in_specs=[
            pl.BlockSpec((1, _BN, 48), lambda i: (0, i, 0)),
            pl.BlockSpec((1, _BN, 48), lambda i: (1, i, 0)),
            pl.BlockSpec((_BN, _O), lambda i: (i, 0)),
        ],
        out_specs=pl.BlockSpec((_BN, _O), lambda i: (i, 0)),
        out_shape=jax.ShapeDtypeStruct((_N, _O), jnp.float32),
    )(s48, s48, si)


def kernel(x, pos, node_attr, edge_index, edge_attr, batch, W_q, W_si,
           W1_k, W2_k, W1_v, W2_v, W_dot):
    wsi2 = jnp.transpose(W_si, (1, 0, 2)).reshape(_A * _C, _O)
    src = edge_index[0]
    dst = edge_index[1]
    ea_t = jnp.transpose(edge_attr)
    w1kv = jnp.concatenate([W1_k, W1_v], axis=1)
    w1e = w1kv[:_NB]
    w1a = w1kv[_NB:]
    tt, si = _node_prep(x, node_attr, pos, W_q, W_dot, wsi2)
    gs, gd = _build_sc_gather()(src, dst, tt)
    out48 = _edge_pass(gs, gd, ea_t, w1e, w1a, W2_k, W2_v)
    s48 = _build_sc_scatter()(dst, out48)
    return _combine(s48, si)


# speedup vs baseline: 1.3475x; 1.0498x over previous
import functools
import numpy as np
import jax
import jax.numpy as jnp
from jax import lax
from jax.experimental import pallas as pl
from jax.experimental.pallas import tpu as pltpu
from jax.experimental.pallas import tpu_sc as plsc

_N = 10000
_E = 160000
_C = 32
_A = 8
_O = 32
_Q = 16
_K = 16
_NB = 8
_EA = 16
_MAX_R = 6.0
_SILU_NORM = 1.6768
_EMB_C = 1.14136 * float(np.exp(2.0))
_SQRT_NB = float(np.sqrt(_NB))
_INV_S24 = 1.0 / float(np.sqrt(_NB + _EA))
_INV_S128 = 1.0 / float(np.sqrt(128.0))
_INV_S32 = 1.0 / float(np.sqrt(_C))
_QD_SCALE = 1.0 / (_C * float(np.sqrt(_Q * _K)))
_SI_SCALE = 1.0 / float(np.sqrt(_C * _A))

_BN = 1000
_BE = 1280

_NW = 32
_EPW = _E // _NW
_GCH = 1000
_NCH = _EPW // _GCH
_NPT = _N // 16


def _sus(x):
    safe = jnp.where(x > 0.0, x, 1.0)
    return jnp.where(x > 0.0, jnp.exp(-1.0 / safe), 0.0)


def _silu(x):
    return x / (1.0 + jnp.exp(-x))


def _node_prep_body(x_ref, na_ref, pos_ref, wq_ref, wdot_ref, wsi_ref,
                    t_ref, si_ref):
    x = x_ref[...]
    na = na_ref[...]
    wqd = jnp.dot(wq_ref[...], wdot_ref[...], preferred_element_type=jnp.float32)
    qd = jnp.dot(x, wqd, preferred_element_type=jnp.float32) * _QD_SCALE
    t_ref[...] = jnp.concatenate(
        [x, qd, pos_ref[...], jnp.zeros((_BN, 128 - _C - _K - 3), jnp.float32)],
        axis=1)
    xa = jnp.concatenate([x * na[:, a:a + 1] for a in range(_A)], axis=1)
    si_ref[...] = jnp.dot(xa, wsi_ref[...], preferred_element_type=jnp.float32) * _SI_SCALE


def _node_prep(x, node_attr, pos, W_q, W_dot, wsi2):
    grid = (_N // _BN,)
    return pl.pallas_call(
        _node_prep_body,
        grid=grid,
        in_specs=[
            pl.BlockSpec((_BN, _C), lambda i: (i, 0)),
            pl.BlockSpec((_BN, _A), lambda i: (i, 0)),
            pl.BlockSpec((_BN, 3), lambda i: (i, 0)),
            pl.BlockSpec((_C, _Q), lambda i: (0, 0)),
            pl.BlockSpec((_Q, _K), lambda i: (0, 0)),
            pl.BlockSpec((_A * _C, _O), lambda i: (0, 0)),
        ],
        out_specs=[
            pl.BlockSpec((_BN, 128), lambda i: (i, 0)),
            pl.BlockSpec((_BN, _O), lambda i: (i, 0)),
        ],
        out_shape=[
            jax.ShapeDtypeStruct((_N, 128), jnp.float32),
            jax.ShapeDtypeStruct((_N, _O), jnp.float32),
        ],
    )(x, node_attr, pos, W_q, W_dot, wsi2)


def _pick_chunk(epw, cap=440):
  for g in range(cap, 7, -8):
    if epw % g == 0 and g % 8 == 0:
      return g
  raise ValueError(epw)


@functools.cache
def _build_sc_gather(e0, ne):
  epw = ne // _NW
  gch = _pick_chunk(epw)
  nch = epw // gch
  mesh = plsc.VectorSubcoreMesh(core_axis_name="c", subcore_axis_name="s")

  @functools.partial(
    pl.kernel,
    mesh=mesh,
    out_type=(
        jax.ShapeDtypeStruct((ne, 128), jnp.float32),
        jax.ShapeDtypeStruct((ne, 128), jnp.float32),
    ),
    scratch_types=[
        pltpu.VMEM((epw,), jnp.int32),
        pltpu.VMEM((epw,), jnp.int32),
        pltpu.VMEM((gch, 128), jnp.float32),
        pltpu.VMEM((gch, 128), jnp.float32),
        pltpu.SemaphoreType.DMA,
        pltpu.SemaphoreType.DMA,
    ],
    compiler_params=pltpu.CompilerParams(needs_layout_passes=False),
  )
  def _sc_gather(src_h, dst_h, tt, gs_out, gd_out, src_v, dst_v, sbuf, dbuf,
                 sem1, sem2):
    wid = lax.axis_index("s") * 2 + lax.axis_index("c")
    base = wid * epw
    pltpu.sync_copy(src_h.at[pl.ds(e0 + base, epw)], src_v)
    pltpu.sync_copy(dst_h.at[pl.ds(e0 + base, epw)], dst_v)

    def chunk_body(ci, carry):
        off = ci * gch
        cp1 = pltpu.async_copy(tt.at[src_v.at[pl.ds(off, gch)]], sbuf, sem1)
        cp2 = pltpu.async_copy(tt.at[dst_v.at[pl.ds(off, gch)]], dbuf, sem2)
        cp1.wait()
        pltpu.sync_copy(sbuf, gs_out.at[pl.ds(base + off, gch)])
        cp2.wait()
        pltpu.sync_copy(dbuf, gd_out.at[pl.ds(base + off, gch)])
        return carry

    lax.fori_loop(0, nch, chunk_body, 0)

  return _sc_gather


def _edge_body(gs_ref, gd_ref, eat_ref, w1e_ref, w1a_ref, w2k_ref,
               w2v_ref, out_ref):
    f32 = jnp.float32
    bf16 = jnp.bfloat16
    gs = gs_ref[...]
    gd = gd_ref[...]

    px_r = lax.broadcasted_iota(jnp.int32, (128, _C), 0)
    px_c = lax.broadcasted_iota(jnp.int32, (128, _C), 1)
    Px = (px_r == px_c).astype(f32)
    pq_r = lax.broadcasted_iota(jnp.int32, (128, _K), 0)
    pq_c = lax.broadcasted_iota(jnp.int32, (128, _K), 1)
    Pq = (pq_r == pq_c + _C).astype(f32)
    pp_r = lax.broadcasted_iota(jnp.int32, (128, 1), 0)
    Pp = ((pp_r >= _C + _K) & (pp_r < _C + _K + 3)).astype(f32)

    xs = jnp.dot(gs, Px, preferred_element_type=f32)
    qd = jnp.dot(gd, Pq, preferred_element_type=f32)
    dp = gs - gd
    l2 = jnp.dot(dp * dp, Pp, preferred_element_type=f32) + 1e-24
    elen = jnp.sqrt(l2)

    jcol = lax.broadcasted_iota(jnp.int32, (_BE, _NB), 1).astype(f32)
    vals = (jcol + 1.0) * (_MAX_R / (_NB + 1))
    diff = (elen - vals) * ((_NB + 1) / _MAX_R)
    dd = 1.0 - diff * diff
    inside = dd > 0.0
    dd_safe = jnp.where(inside, dd, 1.0)
    emb = jnp.where(inside,
                    (_EMB_C * _SQRT_NB) * jnp.exp(-2.0 / dd_safe), 0.0)

    pre = (jnp.dot(emb, w1e_ref[...], preferred_element_type=f32)
           + lax.dot_general(eat_ref[...], w1a_ref[...],
                             (((0,), (0,)), ((), ())),
                             preferred_element_type=f32))
    h = _silu(pre * _INV_S24) * _SILU_NORM
    hk = h[:, 0:128]
    hv = h[:, 128:256]
    wk2 = jnp.dot(hk, w2k_ref[...], preferred_element_type=f32) * _INV_S128
    wv2 = jnp.dot(hv.astype(bf16), w2v_ref[...].astype(bf16),
                  preferred_element_type=f32) * _INV_S128

    rk_r = lax.broadcasted_iota(jnp.int32, (_C, _C * _K), 0)
    rk_c = lax.broadcasted_iota(jnp.int32, (_C, _C * _K), 1)
    Rk = (rk_c // _K == rk_r).astype(f32)
    sk_r = lax.broadcasted_iota(jnp.int32, (_C * _K, _K), 0)
    sk_c = lax.broadcasted_iota(jnp.int32, (_C * _K, _K), 1)
    Sk = (sk_r % _K == sk_c).astype(f32)
    xs_k = jnp.dot(xs, Rk, preferred_element_type=f32)
    kraw = jnp.dot(wk2 * xs_k, Sk, preferred_element_type=f32)

    rv_r = lax.broadcasted_iota(jnp.int32, (_C, _C * _O), 0)
    rv_c = lax.broadcasted_iota(jnp.int32, (_C, _C * _O), 1)
    Rv = (rv_c // _O == rv_r).astype(bf16)
    sv_r = lax.broadcasted_iota(jnp.int32, (_C * _O, _O), 0)
    sv_c = lax.broadcasted_iota(jnp.int32, (_C * _O, _O), 1)
    Sv = (sv_r % _O == sv_c).astype(bf16)
    xs_v = jnp.dot(xs.astype(bf16), Rv, preferred_element_type=f32)
    vraw = jnp.dot((wv2 * xs_v).astype(bf16), Sv,
                   preferred_element_type=f32)

    temp = jnp.sum(qd * kraw, axis=1, keepdims=True)
    ewc = _sus(10.0 * (1.0 - elen * (1.0 / _MAX_R)))
    t2 = ewc * temp
    expv = jnp.exp(t2)
    sexp = jnp.exp(0.5 * t2)
    num = sexp * vraw * _INV_S32
    out_ref[...] = jnp.concatenate(
        [num, expv, jnp.zeros((_BE, 15), f32)], axis=1)


def _edge_pass(gs, gd, ea_t, w1e, w1a, W2_k, W2_v):
    ne = gs.shape[0]
    grid = (ne // _BE,)
    return pl.pallas_call(
        _edge_body,
        grid=grid,
        in_specs=[
            pl.BlockSpec((_BE, 128), lambda i: (i, 0)),
            pl.BlockSpec((_BE, 128), lambda i: (i, 0)),
            pl.BlockSpec((_EA, _BE), lambda i: (0, i)),
            pl.BlockSpec((_NB, 256), lambda i: (0, 0)),
            pl.BlockSpec((_EA, 256), lambda i: (0, 0)),
            pl.BlockSpec((128, _C * _K), lambda i: (0, 0)),
            pl.BlockSpec((128, _C * _O), lambda i: (0, 0)),
        ],
        out_specs=pl.BlockSpec((_BE, 48), lambda i: (i, 0)),
        out_shape=jax.ShapeDtypeStruct((ne, 48), jnp.float32),
    )(gs, gd, ea_t, w1e, w1a, W2_k, W2_v)


@functools.cache
def _build_sc_scatter(na, nb):
  epwa = na // _NW
  epwb = nb // _NW
  gcha = epwa // 5
  gchb = epwb // 5
  mesh = plsc.VectorSubcoreMesh(core_axis_name="c", subcore_axis_name="s")

  @functools.partial(
    pl.kernel,
    mesh=mesh,
    out_type=jax.ShapeDtypeStruct((2, _N, 48), jnp.float32),
    scratch_types=[
        pltpu.VMEM((gchb,), jnp.int32),
        pltpu.VMEM((gchb, 48), jnp.float32),
        pltpu.VMEM((_NPT, 48), jnp.float32),
        pltpu.VMEM_SHARED((_N, 48), jnp.float32),
        pltpu.SemaphoreType.DMA,
    ],
    compiler_params=pltpu.CompilerParams(needs_layout_passes=False, use_tc_tiling_on_sc=False),
  )
  def _sc_scatter(dst_h, rows_a, rows_b, out_h, dstc, rowsv, zb, table, sem):
    cid = lax.axis_index("c")
    sid = lax.axis_index("s")
    wid = sid * 2 + cid

    zero16 = jnp.zeros((16,), jnp.float32)

    def zb_body(i, carry):
        r = i // 3
        c = (i % 3) * 16
        zb[r, pl.ds(c, 16)] = zero16
        return carry

    lax.fori_loop(0, _NPT * 3, zb_body, 0)
    pltpu.sync_copy(zb, table.at[pl.ds(sid * _NPT, _NPT)])
    plsc.subcore_barrier()

    def chunk_a(ci, carry):
        off = wid * epwa + ci * gcha
        pltpu.sync_copy(dst_h.at[pl.ds(off, gcha)], dstc.at[pl.ds(0, gcha)])
        pltpu.sync_copy(rows_a.at[pl.ds(off, gcha)],
                        rowsv.at[pl.ds(0, gcha)])
        pltpu.sync_copy(rowsv.at[pl.ds(0, gcha)],
                        table.at[dstc.at[pl.ds(0, gcha)]], add=True)
        return carry

    lax.fori_loop(0, 5, chunk_a, 0)

    def chunk_b(ci, carry):
        off = wid * epwb + ci * gchb
        pltpu.sync_copy(dst_h.at[pl.ds(na + off, gchb)], dstc)
        pltpu.sync_copy(rows_b.at[pl.ds(off, gchb)], rowsv)
        pltpu.sync_copy(rowsv, table.at[dstc], add=True)
        return carry

    lax.fori_loop(0, 5, chunk_b, 0)
    plsc.subcore_barrier()
    pltpu.sync_copy(table.at[pl.ds(sid * _NPT, _NPT)],
                    out_h.at[cid, pl.ds(sid * _NPT, _NPT)])

  return _sc_scatter


def _combine_body(a0_ref, a1_ref, si_ref, out_ref):
    s = a0_ref[0] + a1_ref[0]
    z = s[:, 32:33]
    zz = jnp.where(z == 0.0, 1.0, z)
    out_ref[...] = si_ref[...] + s[:, 0:_O] * lax.rsqrt(zz)


def _combine(s48, si):
    grid = (_N // _BN,)
    return pl.pallas_call(
        _combine_body,
        grid=grid,
        in_specs=[
            pl.BlockSpec((1, _BN, 48), lambda i: (0, i, 0)),
            pl.BlockSpec((1, _BN, 48), lambda i: (1, i, 0)),
            pl.BlockSpec((_BN, _O), lambda i: (i, 0)),
        ],
        out_specs=pl.BlockSpec((_BN, _O), lambda i: (i, 0)),
        out_shape=jax.ShapeDtypeStruct((_N, _O), jnp.float32),
    )(s48, s48, si)


def kernel(x, pos, node_attr, edge_index, edge_attr, batch, W_q, W_si,
           W1_k, W2_k, W1_v, W2_v, W_dot):
    wsi2 = jnp.transpose(W_si, (1, 0, 2)).reshape(_A * _C, _O)
    src = edge_index[0]
    dst = edge_index[1]
    ea_t = jnp.transpose(edge_attr)
    w1kv = jnp.concatenate([W1_k, W1_v], axis=1)
    w1e = w1kv[:_NB]
    w1a = w1kv[_NB:]
    tt, si = _node_prep(x, node_attr, pos, W_q, W_dot, wsi2)
    na = 62 * _BE
    nb = _E - na
    gs_a, gd_a = _build_sc_gather(0, na)(src, dst, tt)
    gs_b, gd_b = _build_sc_gather(na, nb)(src, dst, tt)
    out_a = _edge_pass(gs_a, gd_a, ea_t[:, 0:na], w1e, w1a, W2_k, W2_v)
    out_b = _edge_pass(gs_b, gd_b, ea_t[:, na:_E], w1e, w1a, W2_k, W2_v)
    s48 = _build_sc_scatter(na, nb)(dst, out_a, out_b)
    return _combine(s48, si)


# speedup vs baseline: 1.3478x; 1.0002x over previous
import functools
import numpy as np
import jax
import jax.numpy as jnp
from jax import lax
from jax.experimental import pallas as pl
from jax.experimental.pallas import tpu as pltpu
from jax.experimental.pallas import tpu_sc as plsc

_N = 10000
_E = 160000
_C = 32
_A = 8
_O = 32
_Q = 16
_K = 16
_NB = 8
_EA = 16
_MAX_R = 6.0
_SILU_NORM = 1.6768
_EMB_C = 1.14136 * float(np.exp(2.0))
_SQRT_NB = float(np.sqrt(_NB))
_INV_S24 = 1.0 / float(np.sqrt(_NB + _EA))
_INV_S128 = 1.0 / float(np.sqrt(128.0))
_INV_S32 = 1.0 / float(np.sqrt(_C))
_QD_SCALE = 1.0 / (_C * float(np.sqrt(_Q * _K)))
_SI_SCALE = 1.0 / float(np.sqrt(_C * _A))

_BN = 1000
_BE = 1280

_NW = 32
_EPW = _E // _NW
_GCH = 1000
_NCH = _EPW // _GCH
_NPT = _N // 16


def _sus(x):
    safe = jnp.where(x > 0.0, x, 1.0)
    return jnp.where(x > 0.0, jnp.exp(-1.0 / safe), 0.0)


def _silu(x):
    return x / (1.0 + jnp.exp(-x))


def _node_prep_body(x_ref, na_ref, pos_ref, wq_ref, wdot_ref, wsi_ref,
                    t_ref, si_ref):
    x = x_ref[...]
    na = na_ref[...]
    wqd = jnp.dot(wq_ref[...], wdot_ref[...], preferred_element_type=jnp.float32)
    qd = jnp.dot(x, wqd, preferred_element_type=jnp.float32) * _QD_SCALE
    t_ref[...] = jnp.concatenate(
        [x, qd, pos_ref[...], jnp.zeros((_BN, 128 - _C - _K - 3), jnp.float32)],
        axis=1)
    xa = jnp.concatenate([x * na[:, a:a + 1] for a in range(_A)], axis=1)
    si_ref[...] = jnp.dot(xa, wsi_ref[...], preferred_element_type=jnp.float32) * _SI_SCALE


def _node_prep(x, node_attr, pos, W_q, W_dot, wsi2):
    grid = (_N // _BN,)
    return pl.pallas_call(
        _node_prep_body,
        grid=grid,
        in_specs=[
            pl.BlockSpec((_BN, _C), lambda i: (i, 0)),
            pl.BlockSpec((_BN, _A), lambda i: (i, 0)),
            pl.BlockSpec((_BN, 3), lambda i: (i, 0)),
            pl.BlockSpec((_C, _Q), lambda i: (0, 0)),
            pl.BlockSpec((_Q, _K), lambda i: (0, 0)),
            pl.BlockSpec((_A * _C, _O), lambda i: (0, 0)),
        ],
        out_specs=[
            pl.BlockSpec((_BN, 128), lambda i: (i, 0)),
            pl.BlockSpec((_BN, _O), lambda i: (i, 0)),
        ],
        out_shape=[
            jax.ShapeDtypeStruct((_N, 128), jnp.float32),
            jax.ShapeDtypeStruct((_N, _O), jnp.float32),
        ],
    )(x, node_attr, pos, W_q, W_dot, wsi2)


def _pick_chunk(epw, cap=440):
  for g in range(cap, 7, -8):
    if epw % g == 0 and g % 8 == 0:
      return g
  raise ValueError(epw)


@functools.cache
def _build_sc_gather(e0, ne):
  epw = ne // _NW
  gch = _pick_chunk(epw)
  nch = epw // gch
  mesh = plsc.VectorSubcoreMesh(core_axis_name="c", subcore_axis_name="s")

  @functools.partial(
    pl.kernel,
    mesh=mesh,
    out_type=(
        jax.ShapeDtypeStruct((ne, 128), jnp.float32),
        jax.ShapeDtypeStruct((ne, 128), jnp.float32),
    ),
    scratch_types=[
        pltpu.VMEM((epw,), jnp.int32),
        pltpu.VMEM((epw,), jnp.int32),
        pltpu.VMEM((gch, 128), jnp.float32),
        pltpu.VMEM((gch, 128), jnp.float32),
        pltpu.SemaphoreType.DMA,
        pltpu.SemaphoreType.DMA,
    ],
    compiler_params=pltpu.CompilerParams(needs_layout_passes=False),
  )
  def _sc_gather(src_h, dst_h, tt, gs_out, gd_out, src_v, dst_v, sbuf, dbuf,
                 sem1, sem2):
    wid = lax.axis_index("s") * 2 + lax.axis_index("c")
    base = wid * epw
    pltpu.sync_copy(src_h.at[pl.ds(e0 + base, epw)], src_v)
    pltpu.sync_copy(dst_h.at[pl.ds(e0 + base, epw)], dst_v)

    def chunk_body(ci, carry):
        off = ci * gch
        cp1 = pltpu.async_copy(tt.at[src_v.at[pl.ds(off, gch)]], sbuf, sem1)
        cp2 = pltpu.async_copy(tt.at[dst_v.at[pl.ds(off, gch)]], dbuf, sem2)
        cp1.wait()
        pltpu.sync_copy(sbuf, gs_out.at[pl.ds(base + off, gch)])
        cp2.wait()
        pltpu.sync_copy(dbuf, gd_out.at[pl.ds(base + off, gch)])
        return carry

    lax.fori_loop(0, nch, chunk_body, 0)

  return _sc_gather


def _edge_body(gs_ref, gd_ref, eat_ref, w1e_ref, w1a_ref, w2k_ref,
               w2v_ref, out_ref):
    f32 = jnp.float32
    bf16 = jnp.bfloat16
    gs = gs_ref[...]
    gd = gd_ref[...]

    px_r = lax.broadcasted_iota(jnp.int32, (128, _C), 0)
    px_c = lax.broadcasted_iota(jnp.int32, (128, _C), 1)
    Px = (px_r == px_c).astype(f32)
    pq_r = lax.broadcasted_iota(jnp.int32, (128, _K), 0)
    pq_c = lax.broadcasted_iota(jnp.int32, (128, _K), 1)
    Pq = (pq_r == pq_c + _C).astype(f32)
    pp_r = lax.broadcasted_iota(jnp.int32, (128, 1), 0)
    Pp = ((pp_r >= _C + _K) & (pp_r < _C + _K + 3)).astype(f32)

    xs = jnp.dot(gs, Px, preferred_element_type=f32)
    qd = jnp.dot(gd, Pq, preferred_element_type=f32)
    dp = gs - gd
    l2 = jnp.dot(dp * dp, Pp, preferred_element_type=f32) + 1e-24
    elen = jnp.sqrt(l2)

    jcol = lax.broadcasted_iota(jnp.int32, (_BE, _NB), 1).astype(f32)
    vals = (jcol + 1.0) * (_MAX_R / (_NB + 1))
    diff = (elen - vals) * ((_NB + 1) / _MAX_R)
    dd = 1.0 - diff * diff
    inside = dd > 0.0
    dd_safe = jnp.where(inside, dd, 1.0)
    emb = jnp.where(inside,
                    (_EMB_C * _SQRT_NB) * jnp.exp(-2.0 / dd_safe), 0.0)

    pre = (jnp.dot(emb, w1e_ref[...], preferred_element_type=f32)
           + lax.dot_general(eat_ref[...], w1a_ref[...],
                             (((0,), (0,)), ((), ())),
                             preferred_element_type=f32))
    h = _silu(pre * _INV_S24) * _SILU_NORM
    hk = h[:, 0:128]
    hv = h[:, 128:256]
    wk2 = jnp.dot(hk, w2k_ref[...], preferred_element_type=f32) * _INV_S128
    wv2 = jnp.dot(hv.astype(bf16), w2v_ref[...].astype(bf16),
                  preferred_element_type=f32) * _INV_S128

    rk_r = lax.broadcasted_iota(jnp.int32, (_C, _C * _K), 0)
    rk_c = lax.broadcasted_iota(jnp.int32, (_C, _C * _K), 1)
    Rk = (rk_c // _K == rk_r).astype(f32)
    sk_r = lax.broadcasted_iota(jnp.int32, (_C * _K, _K), 0)
    sk_c = lax.broadcasted_iota(jnp.int32, (_C * _K, _K), 1)
    Sk = (sk_r % _K == sk_c).astype(f32)
    xs_k = jnp.dot(xs, Rk, preferred_element_type=f32)
    kraw = jnp.dot(wk2 * xs_k, Sk, preferred_element_type=f32)

    rv_r = lax.broadcasted_iota(jnp.int32, (_C, _C * _O), 0)
    rv_c = lax.broadcasted_iota(jnp.int32, (_C, _C * _O), 1)
    Rv = (rv_c // _O == rv_r).astype(bf16)
    sv_r = lax.broadcasted_iota(jnp.int32, (_C * _O, _O), 0)
    sv_c = lax.broadcasted_iota(jnp.int32, (_C * _O, _O), 1)
    Sv = (sv_r % _O == sv_c).astype(bf16)
    xs_v = jnp.dot(xs.astype(bf16), Rv, preferred_element_type=f32)
    vraw = jnp.dot((wv2 * xs_v).astype(bf16), Sv,
                   preferred_element_type=f32)

    temp = jnp.sum(qd * kraw, axis=1, keepdims=True)
    ewc = _sus(10.0 * (1.0 - elen * (1.0 / _MAX_R)))
    t2 = ewc * temp
    expv = jnp.exp(t2)
    sexp = jnp.exp(0.5 * t2)
    num = sexp * vraw * _INV_S32
    out_ref[...] = jnp.concatenate(
        [num, expv, jnp.zeros((_BE, 15), f32)], axis=1)


def _edge_pass(gs, gd, ea_t, w1e, w1a, W2_k, W2_v):
    ne = gs.shape[0]
    grid = (ne // _BE,)
    return pl.pallas_call(
        _edge_body,
        grid=grid,
        in_specs=[
            pl.BlockSpec((_BE, 128), lambda i: (i, 0)),
            pl.BlockSpec((_BE, 128), lambda i: (i, 0)),
            pl.BlockSpec((_EA, _BE), lambda i: (0, i)),
            pl.BlockSpec((_NB, 256), lambda i: (0, 0)),
            pl.BlockSpec((_EA, 256), lambda i: (0, 0)),
            pl.BlockSpec((128, _C * _K), lambda i: (0, 0)),
            pl.BlockSpec((128, _C * _O), lambda i: (0, 0)),
        ],
        out_specs=pl.BlockSpec((_BE, 48), lambda i: (i, 0)),
        out_shape=jax.ShapeDtypeStruct((ne, 48), jnp.float32),
    )(gs, gd, ea_t, w1e, w1a, W2_k, W2_v)


@functools.cache
def _build_sc_scatter(na, nb):
  epwa = na // _NW
  epwb = nb // _NW
  gcha = epwa // 5
  gchb = epwb // 5
  mesh = plsc.VectorSubcoreMesh(core_axis_name="c", subcore_axis_name="s")

  @functools.partial(
    pl.kernel,
    mesh=mesh,
    out_type=jax.ShapeDtypeStruct((2, _N, 48), jnp.float32),
    scratch_types=[
        pltpu.VMEM((gcha,), jnp.int32),
        pltpu.VMEM((gcha, 48), jnp.float32),
        pltpu.VMEM((gchb,), jnp.int32),
        pltpu.VMEM((gchb, 48), jnp.float32),
        pltpu.VMEM((_NPT, 48), jnp.float32),
        pltpu.VMEM_SHARED((_N, 48), jnp.float32),
        pltpu.SemaphoreType.DMA,
    ],
    compiler_params=pltpu.CompilerParams(needs_layout_passes=False, use_tc_tiling_on_sc=False),
  )
  def _sc_scatter(dst_h, rows_a, rows_b, out_h, dstca, rowsva, dstc, rowsv,
                  zb, table, sem):
    cid = lax.axis_index("c")
    sid = lax.axis_index("s")
    wid = sid * 2 + cid

    zero16 = jnp.zeros((16,), jnp.float32)

    def zb_body(i, carry):
        r = i // 3
        c = (i % 3) * 16
        zb[r, pl.ds(c, 16)] = zero16
        return carry

    lax.fori_loop(0, _NPT * 3, zb_body, 0)
    pltpu.sync_copy(zb, table.at[pl.ds(sid * _NPT, _NPT)])
    plsc.subcore_barrier()

    def chunk_a(ci, carry):
        off = wid * epwa + ci * gcha
        pltpu.sync_copy(dst_h.at[pl.ds(off, gcha)], dstca)
        pltpu.sync_copy(rows_a.at[pl.ds(off, gcha)], rowsva)
        pltpu.sync_copy(rowsva, table.at[dstca], add=True)
        return carry

    lax.fori_loop(0, 5, chunk_a, 0)

    def chunk_b(ci, carry):
        off = wid * epwb + ci * gchb
        pltpu.sync_copy(dst_h.at[pl.ds(na + off, gchb)], dstc)
        pltpu.sync_copy(rows_b.at[pl.ds(off, gchb)], rowsv)
        pltpu.sync_copy(rowsv, table.at[dstc], add=True)
        return carry

    lax.fori_loop(0, 5, chunk_b, 0)
    plsc.subcore_barrier()
    pltpu.sync_copy(table.at[pl.ds(sid * _NPT, _NPT)],
                    out_h.at[cid, pl.ds(sid * _NPT, _NPT)])

  return _sc_scatter


def _combine_body(a0_ref, a1_ref, si_ref, out_ref):
    s = a0_ref[0] + a1_ref[0]
    z = s[:, 32:33]
    zz = jnp.where(z == 0.0, 1.0, z)
    out_ref[...] = si_ref[...] + s[:, 0:_O] * lax.rsqrt(zz)


def _combine(s48, si):
    grid = (_N // _BN,)
    return pl.pallas_call(
        _combine_body,
        grid=grid,
        in_specs=[
            pl.BlockSpec((1, _BN, 48), lambda i: (0, i, 0)),
            pl.BlockSpec((1, _BN, 48), lambda i: (1, i, 0)),
            pl.BlockSpec((_BN, _O), lambda i: (i, 0)),
        ],
        out_specs=pl.BlockSpec((_BN, _O), lambda i: (i, 0)),
        out_shape=jax.ShapeDtypeStruct((_N, _O), jnp.float32),
    )(s48, s48, si)


def kernel(x, pos, node_attr, edge_index, edge_attr, batch, W_q, W_si,
           W1_k, W2_k, W1_v, W2_v, W_dot):
    wsi2 = jnp.transpose(W_si, (1, 0, 2)).reshape(_A * _C, _O)
    src = edge_index[0]
    dst = edge_index[1]
    ea_t = jnp.transpose(edge_attr)
    w1kv = jnp.concatenate([W1_k, W1_v], axis=1)
    w1e = w1kv[:_NB]
    w1a = w1kv[_NB:]
    tt, si = _node_prep(x, node_attr, pos, W_q, W_dot, wsi2)
    na = 62 * _BE
    nb = _E - na
    gs_a, gd_a = _build_sc_gather(0, na)(src, dst, tt)
    gs_b, gd_b = _build_sc_gather(na, nb)(src, dst, tt)
    out_a = _edge_pass(gs_a, gd_a, ea_t[:, 0:na], w1e, w1a, W2_k, W2_v)
    out_b = _edge_pass(gs_b, gd_b, ea_t[:, na:_E], w1e, w1a, W2_k, W2_v)
    s48 = _build_sc_scatter(na, nb)(dst, out_a, out_b)
    return _combine(s48, si)


# speedup vs baseline: 1.4148x; 1.0498x over previous
import functools
import numpy as np
import jax
import jax.numpy as jnp
from jax import lax
from jax.experimental import pallas as pl
from jax.experimental.pallas import tpu as pltpu
from jax.experimental.pallas import tpu_sc as plsc

_N = 10000
_E = 160000
_C = 32
_A = 8
_O = 32
_Q = 16
_K = 16
_NB = 8
_EA = 16
_MAX_R = 6.0
_SILU_NORM = 1.6768
_EMB_C = 1.14136 * float(np.exp(2.0))
_SQRT_NB = float(np.sqrt(_NB))
_INV_S24 = 1.0 / float(np.sqrt(_NB + _EA))
_INV_S128 = 1.0 / float(np.sqrt(128.0))
_INV_S32 = 1.0 / float(np.sqrt(_C))
_QD_SCALE = 1.0 / (_C * float(np.sqrt(_Q * _K)))
_SI_SCALE = 1.0 / float(np.sqrt(_C * _A))

_BN = 1000
_BE = 1280

_NW = 32
_EPW = _E // _NW
_GCH = 1000
_NCH = _EPW // _GCH
_NPT = _N // 16


def _sus(x):
    safe = jnp.where(x > 0.0, x, 1.0)
    return jnp.where(x > 0.0, jnp.exp(-1.0 / safe), 0.0)


def _silu(x):
    return x / (1.0 + jnp.exp(-x))


def _node_prep_body(x_ref, na_ref, pos_ref, wq_ref, wdot_ref, wsi_ref,
                    t_ref, si_ref):
    x = x_ref[...]
    na = na_ref[...]
    wqd = jnp.dot(wq_ref[...], wdot_ref[...], preferred_element_type=jnp.float32)
    qd = jnp.dot(x, wqd, preferred_element_type=jnp.float32) * _QD_SCALE
    t_ref[...] = jnp.concatenate(
        [x, qd, pos_ref[...], jnp.zeros((_BN, 128 - _C - _K - 3), jnp.float32)],
        axis=1)
    xa = jnp.concatenate([x * na[:, a:a + 1] for a in range(_A)], axis=1)
    si_ref[...] = jnp.dot(xa, wsi_ref[...], preferred_element_type=jnp.float32) * _SI_SCALE


def _node_prep(x, node_attr, pos, W_q, W_dot, wsi2):
    grid = (_N // _BN,)
    return pl.pallas_call(
        _node_prep_body,
        grid=grid,
        in_specs=[
            pl.BlockSpec((_BN, _C), lambda i: (i, 0)),
            pl.BlockSpec((_BN, _A), lambda i: (i, 0)),
            pl.BlockSpec((_BN, 3), lambda i: (i, 0)),
            pl.BlockSpec((_C, _Q), lambda i: (0, 0)),
            pl.BlockSpec((_Q, _K), lambda i: (0, 0)),
            pl.BlockSpec((_A * _C, _O), lambda i: (0, 0)),
        ],
        out_specs=[
            pl.BlockSpec((_BN, 128), lambda i: (i, 0)),
            pl.BlockSpec((_BN, _O), lambda i: (i, 0)),
        ],
        out_shape=[
            jax.ShapeDtypeStruct((_N, 128), jnp.float32),
            jax.ShapeDtypeStruct((_N, _O), jnp.float32),
        ],
    )(x, node_attr, pos, W_q, W_dot, wsi2)


def _pick_chunk(epw, cap=440):
  for g in range(cap, 7, -8):
    if epw % g == 0 and g % 8 == 0:
      return g
  raise ValueError(epw)


@functools.cache
def _build_sc_gather(e0, ne):
  epw = ne // _NW
  gch = _pick_chunk(epw)
  nch = epw // gch
  mesh = plsc.VectorSubcoreMesh(core_axis_name="c", subcore_axis_name="s")

  @functools.partial(
    pl.kernel,
    mesh=mesh,
    out_type=(
        jax.ShapeDtypeStruct((ne, 128), jnp.float32),
        jax.ShapeDtypeStruct((ne, 128), jnp.float32),
    ),
    scratch_types=[
        pltpu.VMEM((epw,), jnp.int32),
        pltpu.VMEM((epw,), jnp.int32),
        pltpu.VMEM((gch, 128), jnp.float32),
        pltpu.VMEM((gch, 128), jnp.float32),
        pltpu.SemaphoreType.DMA,
        pltpu.SemaphoreType.DMA,
    ],
    compiler_params=pltpu.CompilerParams(needs_layout_passes=False),
  )
  def _sc_gather(src_h, dst_h, tt, gs_out, gd_out, src_v, dst_v, sbuf, dbuf,
                 sem1, sem2):
    wid = lax.axis_index("s") * 2 + lax.axis_index("c")
    base = wid * epw
    pltpu.sync_copy(src_h.at[pl.ds(e0 + base, epw)], src_v)
    pltpu.sync_copy(dst_h.at[pl.ds(e0 + base, epw)], dst_v)

    def chunk_body(ci, carry):
        off = ci * gch
        cp1 = pltpu.async_copy(tt.at[src_v.at[pl.ds(off, gch)]], sbuf, sem1)
        cp2 = pltpu.async_copy(tt.at[dst_v.at[pl.ds(off, gch)]], dbuf, sem2)
        cp1.wait()
        pltpu.sync_copy(sbuf, gs_out.at[pl.ds(base + off, gch)])
        cp2.wait()
        pltpu.sync_copy(dbuf, gd_out.at[pl.ds(base + off, gch)])
        return carry

    lax.fori_loop(0, nch, chunk_body, 0)

  return _sc_gather


def _edge_body(gs_ref, gd_ref, eat_ref, w1e_ref, w1a_ref, w2k_ref,
               w2v_ref, out_ref):
    f32 = jnp.float32
    bf16 = jnp.bfloat16
    gs = gs_ref[...]
    gd = gd_ref[...]

    px_r = lax.broadcasted_iota(jnp.int32, (128, _C), 0)
    px_c = lax.broadcasted_iota(jnp.int32, (128, _C), 1)
    Px = (px_r == px_c).astype(f32)
    pq_r = lax.broadcasted_iota(jnp.int32, (128, _K), 0)
    pq_c = lax.broadcasted_iota(jnp.int32, (128, _K), 1)
    Pq = (pq_r == pq_c + _C).astype(f32)
    pp_r = lax.broadcasted_iota(jnp.int32, (128, 1), 0)
    Pp = ((pp_r >= _C + _K) & (pp_r < _C + _K + 3)).astype(f32)

    xs = jnp.dot(gs, Px, preferred_element_type=f32)
    qd = jnp.dot(gd, Pq, preferred_element_type=f32)
    dp = gs - gd
    l2 = jnp.dot(dp * dp, Pp, preferred_element_type=f32) + 1e-24
    elen = jnp.sqrt(l2)

    jcol = lax.broadcasted_iota(jnp.int32, (_BE, _NB), 1).astype(f32)
    vals = (jcol + 1.0) * (_MAX_R / (_NB + 1))
    diff = (elen - vals) * ((_NB + 1) / _MAX_R)
    dd = 1.0 - diff * diff
    inside = dd > 0.0
    dd_safe = jnp.where(inside, dd, 1.0)
    emb = jnp.where(inside,
                    (_EMB_C * _SQRT_NB) * jnp.exp(-2.0 / dd_safe), 0.0)

    pre = (jnp.dot(emb, w1e_ref[...], preferred_element_type=f32)
           + lax.dot_general(eat_ref[...], w1a_ref[...],
                             (((0,), (0,)), ((), ())),
                             preferred_element_type=f32))
    h = _silu(pre * _INV_S24) * _SILU_NORM
    hk = h[:, 0:128]
    hv = h[:, 128:256]
    wk2 = jnp.dot(hk, w2k_ref[...], preferred_element_type=f32) * _INV_S128
    wv2 = jnp.dot(hv.astype(bf16), w2v_ref[...].astype(bf16),
                  preferred_element_type=f32) * _INV_S128

    rk_r = lax.broadcasted_iota(jnp.int32, (_C, _C * _K), 0)
    rk_c = lax.broadcasted_iota(jnp.int32, (_C, _C * _K), 1)
    Rk = (rk_c // _K == rk_r).astype(f32)
    sk_r = lax.broadcasted_iota(jnp.int32, (_C * _K, _K), 0)
    sk_c = lax.broadcasted_iota(jnp.int32, (_C * _K, _K), 1)
    Sk = (sk_r % _K == sk_c).astype(f32)
    xs_k = jnp.dot(xs, Rk, preferred_element_type=f32)
    kraw = jnp.dot(wk2 * xs_k, Sk, preferred_element_type=f32)

    rv_r = lax.broadcasted_iota(jnp.int32, (_C, _C * _O), 0)
    rv_c = lax.broadcasted_iota(jnp.int32, (_C, _C * _O), 1)
    Rv = (rv_c // _O == rv_r).astype(bf16)
    sv_r = lax.broadcasted_iota(jnp.int32, (_C * _O, _O), 0)
    sv_c = lax.broadcasted_iota(jnp.int32, (_C * _O, _O), 1)
    Sv = (sv_r % _O == sv_c).astype(bf16)
    xs_v = jnp.dot(xs.astype(bf16), Rv, preferred_element_type=f32)
    vraw = jnp.dot((wv2 * xs_v).astype(bf16), Sv,
                   preferred_element_type=f32)

    temp = jnp.sum(qd * kraw, axis=1, keepdims=True)
    ewc = _sus(10.0 * (1.0 - elen * (1.0 / _MAX_R)))
    t2 = ewc * temp
    expv = jnp.exp(t2)
    sexp = jnp.exp(0.5 * t2)
    num = sexp * vraw * _INV_S32
    out_ref[...] = jnp.concatenate(
        [num, expv, jnp.zeros((_BE, 15), f32)], axis=1)


def _edge_pass(gs, gd, ea_t, w1e, w1a, W2_k, W2_v, blk0):
    ne = gs.shape[0]
    grid = (ne // _BE,)
    return pl.pallas_call(
        _edge_body,
        grid=grid,
        in_specs=[
            pl.BlockSpec((_BE, 128), lambda i: (i, 0)),
            pl.BlockSpec((_BE, 128), lambda i: (i, 0)),
            pl.BlockSpec((_EA, _BE), lambda i: (0, i + blk0)),
            pl.BlockSpec((_NB, 256), lambda i: (0, 0)),
            pl.BlockSpec((_EA, 256), lambda i: (0, 0)),
            pl.BlockSpec((128, _C * _K), lambda i: (0, 0)),
            pl.BlockSpec((128, _C * _O), lambda i: (0, 0)),
        ],
        out_specs=pl.BlockSpec((_BE, 48), lambda i: (i, 0)),
        out_shape=jax.ShapeDtypeStruct((ne, 48), jnp.float32),
    )(gs, gd, ea_t, w1e, w1a, W2_k, W2_v)


@functools.cache
def _build_sc_scatterN(spec):
  epws = [ne // _NW for (_, ne) in spec]
  gchs = [_pick_chunk(epw, 504) for epw in epws]
  k = len(spec)
  mesh = plsc.VectorSubcoreMesh(core_axis_name="c", subcore_axis_name="s")

  scratch = []
  for i in range(k):
    scratch.append(pltpu.VMEM((gchs[i],), jnp.int32))
    scratch.append(pltpu.VMEM((gchs[i], 48), jnp.float32))
  scratch += [
      pltpu.VMEM((_NPT, 48), jnp.float32),
      pltpu.VMEM_SHARED((_N, 48), jnp.float32),
      pltpu.SemaphoreType.DMA,
  ]

  @functools.partial(
    pl.kernel,
    mesh=mesh,
    out_type=jax.ShapeDtypeStruct((2, _N, 48), jnp.float32),
    scratch_types=scratch,
    compiler_params=pltpu.CompilerParams(needs_layout_passes=False, use_tc_tiling_on_sc=False),
  )
  def _sc_scatter(dst_h, *args):
    rows = args[0:k]
    out_h = args[k]
    bufs = args[k + 1:k + 1 + 2 * k]
    zb = args[k + 1 + 2 * k]
    table = args[k + 2 + 2 * k]
    cid = lax.axis_index("c")
    sid = lax.axis_index("s")
    wid = sid * 2 + cid

    zero16 = jnp.zeros((16,), jnp.float32)

    def zb_body(i, carry):
        r = i // 3
        c = (i % 3) * 16
        zb[r, pl.ds(c, 16)] = zero16
        return carry

    lax.fori_loop(0, _NPT * 3, zb_body, 0)
    pltpu.sync_copy(zb, table.at[pl.ds(sid * _NPT, _NPT)])
    plsc.subcore_barrier()

    for i in range(k):
        e0, _ = spec[i]
        epw, gch = epws[i], gchs[i]
        dstc, rowsv = bufs[2 * i], bufs[2 * i + 1]
        rows_h = rows[i]

        def chunk_body(ci, carry, e0=e0, epw=epw, gch=gch, dstc=dstc,
                       rowsv=rowsv, rows_h=rows_h):
            off = wid * epw + ci * gch
            pltpu.sync_copy(dst_h.at[pl.ds(e0 + off, gch)], dstc)
            pltpu.sync_copy(rows_h.at[pl.ds(off, gch)], rowsv)
            pltpu.sync_copy(rowsv, table.at[dstc], add=True)
            return carry

        lax.fori_loop(0, epw // gch, chunk_body, 0)

    plsc.subcore_barrier()
    pltpu.sync_copy(table.at[pl.ds(sid * _NPT, _NPT)],
                    out_h.at[cid, pl.ds(sid * _NPT, _NPT)])

  return _sc_scatter


def _combine_body(a0_ref, a1_ref, b0_ref, b1_ref, si_ref, out_ref):
    s = (a0_ref[0] + a1_ref[0]) + (b0_ref[0] + b1_ref[0])
    z = s[:, 32:33]
    zz = jnp.where(z == 0.0, 1.0, z)
    out_ref[...] = si_ref[...] + s[:, 0:_O] * lax.rsqrt(zz)


def _combine(s48a, s48b, si):
    grid = (_N // _BN,)
    return pl.pallas_call(
        _combine_body,
        grid=grid,
        in_specs=[
            pl.BlockSpec((1, _BN, 48), lambda i: (0, i, 0)),
            pl.BlockSpec((1, _BN, 48), lambda i: (1, i, 0)),
            pl.BlockSpec((1, _BN, 48), lambda i: (0, i, 0)),
            pl.BlockSpec((1, _BN, 48), lambda i: (1, i, 0)),
            pl.BlockSpec((_BN, _O), lambda i: (i, 0)),
        ],
        out_specs=pl.BlockSpec((_BN, _O), lambda i: (i, 0)),
        out_shape=jax.ShapeDtypeStruct((_N, _O), jnp.float32),
    )(s48a, s48a, s48b, s48b, si)


def kernel(x, pos, node_attr, edge_index, edge_attr, batch, W_q, W_si,
           W1_k, W2_k, W1_v, W2_v, W_dot):
    wsi2 = jnp.transpose(W_si, (1, 0, 2)).reshape(_A * _C, _O)
    src = edge_index[0]
    dst = edge_index[1]
    ea_t = jnp.transpose(edge_attr)
    w1kv = jnp.concatenate([W1_k, W1_v], axis=1)
    w1e = w1kv[:_NB]
    w1a = w1kv[_NB:]
    tt, si = _node_prep(x, node_attr, pos, W_q, W_dot, wsi2)
    sizes = (16 * _BE, 36 * _BE, 36 * _BE, 37 * _BE)
    starts = (0, sizes[0], sizes[0] + sizes[1], sizes[0] + sizes[1] + sizes[2])
    outs = []
    for e0, ne in zip(starts, sizes):
        gs_c, gd_c = _build_sc_gather(e0, ne)(src, dst, tt)
        outs.append(_edge_pass(gs_c, gd_c, ea_t, w1e, w1a, W2_k, W2_v,
                               e0 // _BE))
    s48a = _build_sc_scatterN(tuple(zip(starts[:3], sizes[:3])))(
        dst, outs[0], outs[1], outs[2])
    s48b = _build_sc_scatterN(((starts[3], sizes[3]),))(dst, outs[3])
    return _combine(s48a, s48b, si)


# speedup vs baseline: 1.5382x; 1.0872x over previous
import functools
import numpy as np
import jax
import jax.numpy as jnp
from jax import lax
from jax.experimental import pallas as pl
from jax.experimental.pallas import tpu as pltpu
from jax.experimental.pallas import tpu_sc as plsc

_N = 10000
_E = 160000
_C = 32
_A = 8
_O = 32
_Q = 16
_K = 16
_NB = 8
_EA = 16
_MAX_R = 6.0
_SILU_NORM = 1.6768
_EMB_C = 1.14136 * float(np.exp(2.0))
_SQRT_NB = float(np.sqrt(_NB))
_INV_S24 = 1.0 / float(np.sqrt(_NB + _EA))
_INV_S128 = 1.0 / float(np.sqrt(128.0))
_INV_S32 = 1.0 / float(np.sqrt(_C))
_QD_SCALE = 1.0 / (_C * float(np.sqrt(_Q * _K)))
_SI_SCALE = 1.0 / float(np.sqrt(_C * _A))

_BN = 1000
_BE = 1280

_NW = 32
_EPW = _E // _NW
_GCH = 1000
_NCH = _EPW // _GCH
_NPT = _N // 16


def _sus(x):
    safe = jnp.where(x > 0.0, x, 1.0)
    return jnp.where(x > 0.0, jnp.exp(-1.0 / safe), 0.0)


def _silu(x):
    return x / (1.0 + jnp.exp(-x))


def _node_prep_body(x_ref, na_ref, pos_ref, wq_ref, wdot_ref, wsi_ref,
                    t_ref, si_ref):
    x = x_ref[...]
    na = na_ref[...]
    wqd = jnp.dot(wq_ref[...], wdot_ref[...], preferred_element_type=jnp.float32)
    qd = jnp.dot(x, wqd, preferred_element_type=jnp.float32) * _QD_SCALE
    t_ref[...] = jnp.concatenate(
        [x, qd, pos_ref[...], jnp.zeros((_BN, 128 - _C - _K - 3), jnp.float32)],
        axis=1)
    xa = jnp.concatenate([x * na[:, a:a + 1] for a in range(_A)], axis=1)
    si_ref[...] = jnp.dot(xa, wsi_ref[...], preferred_element_type=jnp.float32) * _SI_SCALE


def _node_prep(x, node_attr, pos, W_q, W_dot, wsi2):
    grid = (_N // _BN,)
    return pl.pallas_call(
        _node_prep_body,
        grid=grid,
        in_specs=[
            pl.BlockSpec((_BN, _C), lambda i: (i, 0)),
            pl.BlockSpec((_BN, _A), lambda i: (i, 0)),
            pl.BlockSpec((_BN, 3), lambda i: (i, 0)),
            pl.BlockSpec((_C, _Q), lambda i: (0, 0)),
            pl.BlockSpec((_Q, _K), lambda i: (0, 0)),
            pl.BlockSpec((_A * _C, _O), lambda i: (0, 0)),
        ],
        out_specs=[
            pl.BlockSpec((_BN, 128), lambda i: (i, 0)),
            pl.BlockSpec((_BN, _O), lambda i: (i, 0)),
        ],
        out_shape=[
            jax.ShapeDtypeStruct((_N, 128), jnp.float32),
            jax.ShapeDtypeStruct((_N, _O), jnp.float32),
        ],
    )(x, node_attr, pos, W_q, W_dot, wsi2)


def _pick_chunk(epw, cap=440):
  for g in range(cap, 7, -8):
    if epw % g == 0 and g % 8 == 0:
      return g
  raise ValueError(epw)


@functools.cache
def _build_sc_gather(e0, ne):
  epw = ne // _NW
  gch = _pick_chunk(epw)
  nch = epw // gch
  mesh = plsc.VectorSubcoreMesh(core_axis_name="c", subcore_axis_name="s")

  @functools.partial(
    pl.kernel,
    mesh=mesh,
    out_type=(
        jax.ShapeDtypeStruct((ne, 128), jnp.float32),
        jax.ShapeDtypeStruct((ne, 128), jnp.float32),
    ),
    scratch_types=[
        pltpu.VMEM((epw,), jnp.int32),
        pltpu.VMEM((epw,), jnp.int32),
        pltpu.VMEM((gch, 128), jnp.float32),
        pltpu.VMEM((gch, 128), jnp.float32),
        pltpu.SemaphoreType.DMA,
        pltpu.SemaphoreType.DMA,
    ],
    compiler_params=pltpu.CompilerParams(needs_layout_passes=False),
  )
  def _sc_gather(src_h, dst_h, tt, gs_out, gd_out, src_v, dst_v, sbuf, dbuf,
                 sem1, sem2):
    wid = lax.axis_index("s") * 2 + lax.axis_index("c")
    base = wid * epw
    pltpu.sync_copy(src_h.at[pl.ds(e0 + base, epw)], src_v)
    pltpu.sync_copy(dst_h.at[pl.ds(e0 + base, epw)], dst_v)

    def chunk_body(ci, carry):
        off = ci * gch
        cp1 = pltpu.async_copy(tt.at[src_v.at[pl.ds(off, gch)]], sbuf, sem1)
        cp2 = pltpu.async_copy(tt.at[dst_v.at[pl.ds(off, gch)]], dbuf, sem2)
        cp1.wait()
        pltpu.sync_copy(sbuf, gs_out.at[pl.ds(base + off, gch)])
        cp2.wait()
        pltpu.sync_copy(dbuf, gd_out.at[pl.ds(base + off, gch)])
        return carry

    lax.fori_loop(0, nch, chunk_body, 0)

  return _sc_gather


def _edge_body(gs_ref, gd_ref, eat_ref, w1e_ref, w1a_ref, w2k_ref,
               w2v_ref, out_ref):
    f32 = jnp.float32
    bf16 = jnp.bfloat16
    gs = gs_ref[...]
    gd = gd_ref[...]

    px_r = lax.broadcasted_iota(jnp.int32, (128, _C), 0)
    px_c = lax.broadcasted_iota(jnp.int32, (128, _C), 1)
    Px = (px_r == px_c).astype(f32)
    pq_r = lax.broadcasted_iota(jnp.int32, (128, _K), 0)
    pq_c = lax.broadcasted_iota(jnp.int32, (128, _K), 1)
    Pq = (pq_r == pq_c + _C).astype(f32)
    pp_r = lax.broadcasted_iota(jnp.int32, (128, 1), 0)
    Pp = ((pp_r >= _C + _K) & (pp_r < _C + _K + 3)).astype(f32)

    xs = jnp.dot(gs, Px, preferred_element_type=f32)
    qd = jnp.dot(gd, Pq, preferred_element_type=f32)
    dp = gs - gd
    l2 = jnp.dot(dp * dp, Pp, preferred_element_type=f32) + 1e-24
    elen = jnp.sqrt(l2)

    jcol = lax.broadcasted_iota(jnp.int32, (_BE, _NB), 1).astype(f32)
    vals = (jcol + 1.0) * (_MAX_R / (_NB + 1))
    diff = (elen - vals) * ((_NB + 1) / _MAX_R)
    dd = 1.0 - diff * diff
    inside = dd > 0.0
    dd_safe = jnp.where(inside, dd, 1.0)
    emb = jnp.where(inside,
                    (_EMB_C * _SQRT_NB) * jnp.exp(-2.0 / dd_safe), 0.0)

    pre = (jnp.dot(emb, w1e_ref[...], preferred_element_type=f32)
           + lax.dot_general(eat_ref[...], w1a_ref[...],
                             (((0,), (0,)), ((), ())),
                             preferred_element_type=f32))
    h = _silu(pre * _INV_S24) * _SILU_NORM
    hk = h[:, 0:128]
    hv = h[:, 128:256]
    wk2 = jnp.dot(hk, w2k_ref[...], preferred_element_type=f32) * _INV_S128
    wv2 = jnp.dot(hv.astype(bf16), w2v_ref[...].astype(bf16),
                  preferred_element_type=f32) * _INV_S128

    rk_r = lax.broadcasted_iota(jnp.int32, (_C, _C * _K), 0)
    rk_c = lax.broadcasted_iota(jnp.int32, (_C, _C * _K), 1)
    Rk = (rk_c // _K == rk_r).astype(f32)
    sk_r = lax.broadcasted_iota(jnp.int32, (_C * _K, _K), 0)
    sk_c = lax.broadcasted_iota(jnp.int32, (_C * _K, _K), 1)
    Sk = (sk_r % _K == sk_c).astype(f32)
    xs_k = jnp.dot(xs, Rk, preferred_element_type=f32)
    kraw = jnp.dot(wk2 * xs_k, Sk, preferred_element_type=f32)

    rv_r = lax.broadcasted_iota(jnp.int32, (_C, _C * _O), 0)
    rv_c = lax.broadcasted_iota(jnp.int32, (_C, _C * _O), 1)
    Rv = (rv_c // _O == rv_r).astype(bf16)
    sv_r = lax.broadcasted_iota(jnp.int32, (_C * _O, _O), 0)
    sv_c = lax.broadcasted_iota(jnp.int32, (_C * _O, _O), 1)
    Sv = (sv_r % _O == sv_c).astype(bf16)
    xs_v = jnp.dot(xs.astype(bf16), Rv, preferred_element_type=f32)
    vraw = jnp.dot((wv2 * xs_v).astype(bf16), Sv,
                   preferred_element_type=f32)

    temp = jnp.sum(qd * kraw, axis=1, keepdims=True)
    ewc = _sus(10.0 * (1.0 - elen * (1.0 / _MAX_R)))
    t2 = ewc * temp
    expv = jnp.exp(t2)
    sexp = jnp.exp(0.5 * t2)
    num = sexp * vraw * _INV_S32
    out_ref[...] = jnp.concatenate(
        [num, expv, jnp.zeros((_BE, 95), f32)], axis=1)


def _edge_pass(gs, gd, ea_t, w1e, w1a, W2_k, W2_v, blk0):
    ne = gs.shape[0]
    grid = (ne // _BE,)
    return pl.pallas_call(
        _edge_body,
        grid=grid,
        in_specs=[
            pl.BlockSpec((_BE, 128), lambda i: (i, 0)),
            pl.BlockSpec((_BE, 128), lambda i: (i, 0)),
            pl.BlockSpec((_EA, _BE), lambda i: (0, i + blk0)),
            pl.BlockSpec((_NB, 256), lambda i: (0, 0)),
            pl.BlockSpec((_EA, 256), lambda i: (0, 0)),
            pl.BlockSpec((128, _C * _K), lambda i: (0, 0)),
            pl.BlockSpec((128, _C * _O), lambda i: (0, 0)),
        ],
        out_specs=pl.BlockSpec((_BE, 128), lambda i: (i, 0)),
        out_shape=jax.ShapeDtypeStruct((ne, 128), jnp.float32),
    )(gs, gd, ea_t, w1e, w1a, W2_k, W2_v)


@functools.cache
def _build_sc_scatterN(spec):
  epws = [ne // _NW for (_, ne) in spec]
  budget = 376
  gchs = []
  for epw in epws:
    cap = max(8, (budget // len(epws)) - (budget // len(epws)) % 8)
    gchs.append(_pick_chunk(epw, cap))
  k = len(spec)
  mesh = plsc.VectorSubcoreMesh(core_axis_name="c", subcore_axis_name="s")

  scratch = []
  for i in range(k):
    scratch.append(pltpu.VMEM((gchs[i],), jnp.int32))
    scratch.append(pltpu.VMEM((gchs[i], 128), jnp.float32))
  scratch += [
      pltpu.VMEM((8, 128), jnp.float32),
      pltpu.VMEM_SHARED((_N, 128), jnp.float32),
      pltpu.SemaphoreType.DMA,
  ]

  @functools.partial(
    pl.kernel,
    mesh=mesh,
    out_type=jax.ShapeDtypeStruct((2, _N, 128), jnp.float32),
    scratch_types=scratch,
    compiler_params=pltpu.CompilerParams(needs_layout_passes=False),
  )
  def _sc_scatter(dst_h, *args):
    rows = args[0:k]
    out_h = args[k]
    bufs = args[k + 1:k + 1 + 2 * k]
    zb = args[k + 1 + 2 * k]
    table = args[k + 2 + 2 * k]
    cid = lax.axis_index("c")
    sid = lax.axis_index("s")
    wid = sid * 2 + cid

    zero16 = jnp.zeros((16,), jnp.float32)

    def zb_body(i, carry):
        r = i // 8
        c = (i % 8) * 16
        zb[r, pl.ds(c, 16)] = zero16
        return carry

    lax.fori_loop(0, 8 * 8, zb_body, 0)

    def zt_body(j, carry):
        pltpu.sync_copy(zb, table.at[pl.ds(sid * 632 + j * 8, 8)])
        return carry

    @pl.when(sid < 15)
    def _():
        lax.fori_loop(0, 79, zt_body, 0)

    @pl.when(sid == 15)
    def _():
        lax.fori_loop(0, 65, zt_body, 0)

    plsc.subcore_barrier()

    for i in range(k):
        e0, _ = spec[i]
        epw, gch = epws[i], gchs[i]
        dstc, rowsv = bufs[2 * i], bufs[2 * i + 1]
        rows_h = rows[i]

        def chunk_body(ci, carry, e0=e0, epw=epw, gch=gch, dstc=dstc,
                       rowsv=rowsv, rows_h=rows_h):
            off = wid * epw + ci * gch
            pltpu.sync_copy(dst_h.at[pl.ds(e0 + off, gch)], dstc)
            pltpu.sync_copy(rows_h.at[pl.ds(off, gch)], rowsv)
            pltpu.sync_copy(rowsv, table.at[dstc], add=True)
            return carry

        lax.fori_loop(0, epw // gch, chunk_body, 0)

    plsc.subcore_barrier()

    @pl.when(sid < 15)
    def _():
        pltpu.sync_copy(table.at[pl.ds(sid * 632, 632)],
                        out_h.at[cid, pl.ds(sid * 632, 632)])

    @pl.when(sid == 15)
    def _():
        pltpu.sync_copy(table.at[pl.ds(15 * 632, 520)],
                        out_h.at[cid, pl.ds(15 * 632, 520)])

  return _sc_scatter


def _combine_body(a0_ref, a1_ref, b0_ref, b1_ref, si_ref, out_ref):
    s = (a0_ref[0] + a1_ref[0]) + (b0_ref[0] + b1_ref[0])
    z = s[:, 32:33]
    zz = jnp.where(z == 0.0, 1.0, z)
    out_ref[...] = si_ref[...] + s[:, 0:_O] * lax.rsqrt(zz)


def _combine(s48a, s48b, si):
    grid = (_N // _BN,)
    return pl.pallas_call(
        _combine_body,
        grid=grid,
        in_specs=[
            pl.BlockSpec((1, _BN, 128), lambda i: (0, i, 0)),
            pl.BlockSpec((1, _BN, 128), lambda i: (1, i, 0)),
            pl.BlockSpec((1, _BN, 128), lambda i: (0, i, 0)),
            pl.BlockSpec((1, _BN, 128), lambda i: (1, i, 0)),
            pl.BlockSpec((_BN, _O), lambda i: (i, 0)),
        ],
        out_specs=pl.BlockSpec((_BN, _O), lambda i: (i, 0)),
        out_shape=jax.ShapeDtypeStruct((_N, _O), jnp.float32),
    )(s48a, s48a, s48b, s48b, si)


def kernel(x, pos, node_attr, edge_index, edge_attr, batch, W_q, W_si,
           W1_k, W2_k, W1_v, W2_v, W_dot):
    wsi2 = jnp.transpose(W_si, (1, 0, 2)).reshape(_A * _C, _O)
    src = edge_index[0]
    dst = edge_index[1]
    ea_t = jnp.transpose(edge_attr)
    w1kv = jnp.concatenate([W1_k, W1_v], axis=1)
    w1e = w1kv[:_NB]
    w1a = w1kv[_NB:]
    tt, si = _node_prep(x, node_attr, pos, W_q, W_dot, wsi2)
    sizes = (8 * _BE, 38 * _BE, 38 * _BE, 41 * _BE)
    starts = (0, sizes[0], sizes[0] + sizes[1], sizes[0] + sizes[1] + sizes[2])
    outs = []
    for e0, ne in zip(starts, sizes):
        gs_c, gd_c = _build_sc_gather(e0, ne)(src, dst, tt)
        outs.append(_edge_pass(gs_c, gd_c, ea_t, w1e, w1a, W2_k, W2_v,
                               e0 // _BE))
    s48a = _build_sc_scatterN(tuple(zip(starts[:3], sizes[:3])))(
        dst, outs[0], outs[1], outs[2])
    s48b = _build_sc_scatterN(((starts[3], sizes[3]),))(dst, outs[3])
    return _combine(s48a, s48b, si)


# speedup vs baseline: 1.5382x; 1.0000x over previous
import functools
import numpy as np
import jax
import jax.numpy as jnp
from jax import lax
from jax.experimental import pallas as pl
from jax.experimental.pallas import tpu as pltpu
from jax.experimental.pallas import tpu_sc as plsc

_N = 10000
_E = 160000
_C = 32
_A = 8
_O = 32
_Q = 16
_K = 16
_NB = 8
_EA = 16
_MAX_R = 6.0
_SILU_NORM = 1.6768
_EMB_C = 1.14136 * float(np.exp(2.0))
_SQRT_NB = float(np.sqrt(_NB))
_INV_S24 = 1.0 / float(np.sqrt(_NB + _EA))
_INV_S128 = 1.0 / float(np.sqrt(128.0))
_INV_S32 = 1.0 / float(np.sqrt(_C))
_QD_SCALE = 1.0 / (_C * float(np.sqrt(_Q * _K)))
_SI_SCALE = 1.0 / float(np.sqrt(_C * _A))

_BN = 1000
_BE = 1280

_NW = 32
_EPW = _E // _NW
_GCH = 1000
_NCH = _EPW // _GCH
_NPT = _N // 16


def _sus(x):
    safe = jnp.where(x > 0.0, x, 1.0)
    return jnp.where(x > 0.0, jnp.exp(-1.0 / safe), 0.0)


def _silu(x):
    return x / (1.0 + jnp.exp(-x))


def _node_prep_body(x_ref, na_ref, pos_ref, wq_ref, wdot_ref, wsi_ref,
                    t_ref, si_ref):
    x = x_ref[...]
    na = na_ref[...]
    wqd = jnp.dot(wq_ref[...], wdot_ref[...], preferred_element_type=jnp.float32)
    qd = jnp.dot(x, wqd, preferred_element_type=jnp.float32) * _QD_SCALE
    t_ref[...] = jnp.concatenate(
        [x, qd, pos_ref[...], jnp.zeros((_BN, 128 - _C - _K - 3), jnp.float32)],
        axis=1)
    xa = jnp.concatenate([x * na[:, a:a + 1] for a in range(_A)], axis=1)
    si_ref[...] = jnp.dot(xa, wsi_ref[...], preferred_element_type=jnp.float32) * _SI_SCALE


def _node_prep(x, node_attr, pos, W_q, W_dot, wsi2):
    grid = (_N // _BN,)
    return pl.pallas_call(
        _node_prep_body,
        grid=grid,
        in_specs=[
            pl.BlockSpec((_BN, _C), lambda i: (i, 0)),
            pl.BlockSpec((_BN, _A), lambda i: (i, 0)),
            pl.BlockSpec((_BN, 3), lambda i: (i, 0)),
            pl.BlockSpec((_C, _Q), lambda i: (0, 0)),
            pl.BlockSpec((_Q, _K), lambda i: (0, 0)),
            pl.BlockSpec((_A * _C, _O), lambda i: (0, 0)),
        ],
        out_specs=[
            pl.BlockSpec((_BN, 128), lambda i: (i, 0)),
            pl.BlockSpec((_BN, _O), lambda i: (i, 0)),
        ],
        out_shape=[
            jax.ShapeDtypeStruct((_N, 128), jnp.float32),
            jax.ShapeDtypeStruct((_N, _O), jnp.float32),
        ],
    )(x, node_attr, pos, W_q, W_dot, wsi2)


def _pick_chunk(epw, cap=440):
  for g in range(cap, 7, -8):
    if epw % g == 0 and g % 8 == 0:
      return g
  raise ValueError(epw)


@functools.cache
def _build_sc_gather(e0, ne):
  epw = ne // _NW
  gch = _pick_chunk(epw)
  nch = epw // gch
  mesh = plsc.VectorSubcoreMesh(core_axis_name="c", subcore_axis_name="s")

  @functools.partial(
    pl.kernel,
    mesh=mesh,
    out_type=(
        jax.ShapeDtypeStruct((ne, 128), jnp.float32),
        jax.ShapeDtypeStruct((ne, 128), jnp.float32),
    ),
    scratch_types=[
        pltpu.VMEM((epw,), jnp.int32),
        pltpu.VMEM((epw,), jnp.int32),
        pltpu.VMEM((gch, 128), jnp.float32),
        pltpu.VMEM((gch, 128), jnp.float32),
        pltpu.SemaphoreType.DMA,
        pltpu.SemaphoreType.DMA,
    ],
    compiler_params=pltpu.CompilerParams(needs_layout_passes=False),
  )
  def _sc_gather(src_h, dst_h, tt, gs_out, gd_out, src_v, dst_v, sbuf, dbuf,
                 sem1, sem2):
    wid = lax.axis_index("s") * 2 + lax.axis_index("c")
    base = wid * epw
    pltpu.sync_copy(src_h.at[pl.ds(e0 + base, epw)], src_v)
    pltpu.sync_copy(dst_h.at[pl.ds(e0 + base, epw)], dst_v)

    def chunk_body(ci, carry):
        off = ci * gch
        cp1 = pltpu.async_copy(tt.at[src_v.at[pl.ds(off, gch)]], sbuf, sem1)
        cp2 = pltpu.async_copy(tt.at[dst_v.at[pl.ds(off, gch)]], dbuf, sem2)
        cp1.wait()
        pltpu.sync_copy(sbuf, gs_out.at[pl.ds(base + off, gch)])
        cp2.wait()
        pltpu.sync_copy(dbuf, gd_out.at[pl.ds(base + off, gch)])
        return carry

    lax.fori_loop(0, nch, chunk_body, 0)

  return _sc_gather


def _edge_body(gs_ref, gd_ref, eat_ref, w1e_ref, w1a_ref, w2k_ref,
               w2v_ref, out_ref):
    f32 = jnp.float32
    bf16 = jnp.bfloat16
    gs = gs_ref[...]
    gd = gd_ref[...]

    px_r = lax.broadcasted_iota(jnp.int32, (128, _C), 0)
    px_c = lax.broadcasted_iota(jnp.int32, (128, _C), 1)
    Px = (px_r == px_c).astype(f32)
    pq_r = lax.broadcasted_iota(jnp.int32, (128, _K), 0)
    pq_c = lax.broadcasted_iota(jnp.int32, (128, _K), 1)
    Pq = (pq_r == pq_c + _C).astype(f32)
    pp_r = lax.broadcasted_iota(jnp.int32, (128, 1), 0)
    Pp = ((pp_r >= _C + _K) & (pp_r < _C + _K + 3)).astype(f32)

    xs = jnp.dot(gs, Px, preferred_element_type=f32)
    qd = jnp.dot(gd, Pq, preferred_element_type=f32)
    dp = gs - gd
    l2 = jnp.dot(dp * dp, Pp, preferred_element_type=f32) + 1e-24
    elen = jnp.sqrt(l2)

    jcol = lax.broadcasted_iota(jnp.int32, (_BE, _NB), 1).astype(f32)
    vals = (jcol + 1.0) * (_MAX_R / (_NB + 1))
    diff = (elen - vals) * ((_NB + 1) / _MAX_R)
    dd = 1.0 - diff * diff
    inside = dd > 0.0
    dd_safe = jnp.where(inside, dd, 1.0)
    emb = jnp.where(inside,
                    (_EMB_C * _SQRT_NB) * jnp.exp(-2.0 / dd_safe), 0.0)

    pre = (jnp.dot(emb, w1e_ref[...], preferred_element_type=f32)
           + lax.dot_general(eat_ref[...], w1a_ref[...],
                             (((0,), (0,)), ((), ())),
                             preferred_element_type=f32))
    h = _silu(pre * _INV_S24) * _SILU_NORM
    hk = h[:, 0:128]
    hv = h[:, 128:256]
    wk2 = jnp.dot(hk, w2k_ref[...], preferred_element_type=f32) * _INV_S128
    wv2 = jnp.dot(hv.astype(bf16), w2v_ref[...].astype(bf16),
                  preferred_element_type=f32) * _INV_S128

    rk_r = lax.broadcasted_iota(jnp.int32, (_C, _C * _K), 0)
    rk_c = lax.broadcasted_iota(jnp.int32, (_C, _C * _K), 1)
    Rk = (rk_c // _K == rk_r).astype(f32)
    sk_r = lax.broadcasted_iota(jnp.int32, (_C * _K, _K), 0)
    sk_c = lax.broadcasted_iota(jnp.int32, (_C * _K, _K), 1)
    Sk = (sk_r % _K == sk_c).astype(f32)
    xs_k = jnp.dot(xs, Rk, preferred_element_type=f32)
    kraw = jnp.dot(wk2 * xs_k, Sk, preferred_element_type=f32)

    rv_r = lax.broadcasted_iota(jnp.int32, (_C, _C * _O), 0)
    rv_c = lax.broadcasted_iota(jnp.int32, (_C, _C * _O), 1)
    Rv = (rv_c // _O == rv_r).astype(bf16)
    sv_r = lax.broadcasted_iota(jnp.int32, (_C * _O, _O), 0)
    sv_c = lax.broadcasted_iota(jnp.int32, (_C * _O, _O), 1)
    Sv = (sv_r % _O == sv_c).astype(bf16)
    xs_v = jnp.dot(xs.astype(bf16), Rv, preferred_element_type=f32)
    vraw = jnp.dot((wv2 * xs_v).astype(bf16), Sv,
                   preferred_element_type=f32)

    temp = jnp.sum(qd * kraw, axis=1, keepdims=True)
    ewc = _sus(10.0 * (1.0 - elen * (1.0 / _MAX_R)))
    t2 = ewc * temp
    expv = jnp.exp(t2)
    sexp = jnp.exp(0.5 * t2)
    num = sexp * vraw * _INV_S32
    out_ref[...] = jnp.concatenate(
        [num, expv, jnp.zeros((_BE, 95), f32)], axis=1)


def _edge_pass(gs, gd, ea_t, w1e, w1a, W2_k, W2_v, blk0):
    ne = gs.shape[0]
    grid = (ne // _BE,)
    return pl.pallas_call(
        _edge_body,
        grid=grid,
        in_specs=[
            pl.BlockSpec((_BE, 128), lambda i: (i, 0)),
            pl.BlockSpec((_BE, 128), lambda i: (i, 0)),
            pl.BlockSpec((_EA, _BE), lambda i: (0, i + blk0)),
            pl.BlockSpec((_NB, 256), lambda i: (0, 0)),
            pl.BlockSpec((_EA, 256), lambda i: (0, 0)),
            pl.BlockSpec((128, _C * _K), lambda i: (0, 0)),
            pl.BlockSpec((128, _C * _O), lambda i: (0, 0)),
        ],
        out_specs=pl.BlockSpec((_BE, 128), lambda i: (i, 0)),
        out_shape=jax.ShapeDtypeStruct((ne, 128), jnp.float32),
    )(gs, gd, ea_t, w1e, w1a, W2_k, W2_v)


@functools.cache
def _build_sc_scatterN(spec):
  epws = [ne // _NW for (_, ne) in spec]
  budget = 376
  gchs = []
  for epw in epws:
    cap = max(8, (budget // len(epws)) - (budget // len(epws)) % 8)
    gchs.append(_pick_chunk(epw, cap))
  k = len(spec)
  mesh = plsc.VectorSubcoreMesh(core_axis_name="c", subcore_axis_name="s")

  scratch = []
  for i in range(k):
    scratch.append(pltpu.VMEM((gchs[i],), jnp.int32))
    scratch.append(pltpu.VMEM((gchs[i], 128), jnp.float32))
  scratch += [
      pltpu.VMEM((8, 128), jnp.float32),
      pltpu.VMEM_SHARED((_N, 128), jnp.float32),
      pltpu.SemaphoreType.DMA,
  ]

  @functools.partial(
    pl.kernel,
    mesh=mesh,
    out_type=jax.ShapeDtypeStruct((2, _N, 128), jnp.float32),
    scratch_types=scratch,
    compiler_params=pltpu.CompilerParams(needs_layout_passes=False),
  )
  def _sc_scatter(dst_h, *args):
    rows = args[0:k]
    out_h = args[k]
    bufs = args[k + 1:k + 1 + 2 * k]
    zb = args[k + 1 + 2 * k]
    table = args[k + 2 + 2 * k]
    cid = lax.axis_index("c")
    sid = lax.axis_index("s")
    wid = sid * 2 + cid

    zero16 = jnp.zeros((16,), jnp.float32)

    def zb_body(i, carry):
        r = i // 8
        c = (i % 8) * 16
        zb[r, pl.ds(c, 16)] = zero16
        return carry

    lax.fori_loop(0, 8 * 8, zb_body, 0)

    def zt_body(j, carry):
        pltpu.sync_copy(zb, table.at[pl.ds(sid * 632 + j * 8, 8)])
        return carry

    @pl.when(sid < 15)
    def _():
        lax.fori_loop(0, 79, zt_body, 0)

    @pl.when(sid == 15)
    def _():
        lax.fori_loop(0, 65, zt_body, 0)

    plsc.subcore_barrier()

    for i in range(k):
        e0, _ = spec[i]
        epw, gch = epws[i], gchs[i]
        dstc, rowsv = bufs[2 * i], bufs[2 * i + 1]
        rows_h = rows[i]

        def chunk_body(ci, carry, e0=e0, epw=epw, gch=gch, dstc=dstc,
                       rowsv=rowsv, rows_h=rows_h):
            off = wid * epw + ci * gch
            pltpu.sync_copy(dst_h.at[pl.ds(e0 + off, gch)], dstc)
            pltpu.sync_copy(rows_h.at[pl.ds(off, gch)], rowsv)
            pltpu.sync_copy(rowsv, table.at[dstc], add=True)
            return carry

        lax.fori_loop(0, epw // gch, chunk_body, 0)

    plsc.subcore_barrier()

    @pl.when(sid < 15)
    def _():
        pltpu.sync_copy(table.at[pl.ds(sid * 632, 632)],
                        out_h.at[cid, pl.ds(sid * 632, 632)])

    @pl.when(sid == 15)
    def _():
        pltpu.sync_copy(table.at[pl.ds(15 * 632, 520)],
                        out_h.at[cid, pl.ds(15 * 632, 520)])

  return _sc_scatter


def _combine_body(a0_ref, a1_ref, b0_ref, b1_ref, c0_ref, c1_ref, si_ref,
                  out_ref):
    s = ((a0_ref[0] + a1_ref[0]) + (b0_ref[0] + b1_ref[0])
         + (c0_ref[0] + c1_ref[0]))
    z = s[:, 32:33]
    zz = jnp.where(z == 0.0, 1.0, z)
    out_ref[...] = si_ref[...] + s[:, 0:_O] * lax.rsqrt(zz)


def _combine(s48a, s48b, s48c, si):
    grid = (_N // _BN,)
    return pl.pallas_call(
        _combine_body,
        grid=grid,
        in_specs=[
            pl.BlockSpec((1, _BN, 128), lambda i: (0, i, 0)),
            pl.BlockSpec((1, _BN, 128), lambda i: (1, i, 0)),
            pl.BlockSpec((1, _BN, 128), lambda i: (0, i, 0)),
            pl.BlockSpec((1, _BN, 128), lambda i: (1, i, 0)),
            pl.BlockSpec((1, _BN, 128), lambda i: (0, i, 0)),
            pl.BlockSpec((1, _BN, 128), lambda i: (1, i, 0)),
            pl.BlockSpec((_BN, _O), lambda i: (i, 0)),
        ],
        out_specs=pl.BlockSpec((_BN, _O), lambda i: (i, 0)),
        out_shape=jax.ShapeDtypeStruct((_N, _O), jnp.float32),
    )(s48a, s48a, s48b, s48b, s48c, s48c, si)


def kernel(x, pos, node_attr, edge_index, edge_attr, batch, W_q, W_si,
           W1_k, W2_k, W1_v, W2_v, W_dot):
    wsi2 = jnp.transpose(W_si, (1, 0, 2)).reshape(_A * _C, _O)
    src = edge_index[0]
    dst = edge_index[1]
    ea_t = jnp.transpose(edge_attr)
    w1kv = jnp.concatenate([W1_k, W1_v], axis=1)
    w1e = w1kv[:_NB]
    w1a = w1kv[_NB:]
    tt, si = _node_prep(x, node_attr, pos, W_q, W_dot, wsi2)
    sizes = (8 * _BE, 46 * _BE, 50 * _BE, 21 * _BE)
    starts = (0, sizes[0], sizes[0] + sizes[1], sizes[0] + sizes[1] + sizes[2])
    outs = []
    for e0, ne in zip(starts, sizes):
        gs_c, gd_c = _build_sc_gather(e0, ne)(src, dst, tt)
        outs.append(_edge_pass(gs_c, gd_c, ea_t, w1e, w1a, W2_k, W2_v,
                               e0 // _BE))
    s48a = _build_sc_scatterN(tuple(zip(starts[:2], sizes[:2])))(
        dst, outs[0], outs[1])
    s48b = _build_sc_scatterN(((starts[2], sizes[2]),))(dst, outs[2])
    s48c = _build_sc_scatterN(((starts[3], sizes[3]),))(dst, outs[3])
    return _combine(s48a, s48b, s48c, si)


# speedup vs baseline: 1.5705x; 1.0209x over previous
import functools
import numpy as np
import jax
import jax.numpy as jnp
from jax import lax
from jax.experimental import pallas as pl
from jax.experimental.pallas import tpu as pltpu
from jax.experimental.pallas import tpu_sc as plsc

_N = 10000
_E = 160000
_C = 32
_A = 8
_O = 32
_Q = 16
_K = 16
_NB = 8
_EA = 16
_MAX_R = 6.0
_SILU_NORM = 1.6768
_EMB_C = 1.14136 * float(np.exp(2.0))
_SQRT_NB = float(np.sqrt(_NB))
_INV_S24 = 1.0 / float(np.sqrt(_NB + _EA))
_INV_S128 = 1.0 / float(np.sqrt(128.0))
_INV_S32 = 1.0 / float(np.sqrt(_C))
_QD_SCALE = 1.0 / (_C * float(np.sqrt(_Q * _K)))
_SI_SCALE = 1.0 / float(np.sqrt(_C * _A))

_BN = 1000
_BE = 1280

_NW = 32
_EPW = _E // _NW
_GCH = 1000
_NCH = _EPW // _GCH
_NPT = _N // 16


def _sus(x):
    safe = jnp.where(x > 0.0, x, 1.0)
    return jnp.where(x > 0.0, jnp.exp(-1.0 / safe), 0.0)


def _silu(x):
    return x / (1.0 + jnp.exp(-x))


def _prep_table_body(x_ref, pos_ref, wq_ref, wdot_ref, t_ref):
    x = x_ref[...]
    wqd = jnp.dot(wq_ref[...], wdot_ref[...], preferred_element_type=jnp.float32)
    qd = jnp.dot(x, wqd, preferred_element_type=jnp.float32) * _QD_SCALE
    t_ref[...] = jnp.concatenate(
        [x, qd, pos_ref[...], jnp.zeros((_BN, 128 - _C - _K - 3), jnp.float32)],
        axis=1)


def _prep_table(x, pos, W_q, W_dot):
    grid = (_N // _BN,)
    return pl.pallas_call(
        _prep_table_body,
        grid=grid,
        in_specs=[
            pl.BlockSpec((_BN, _C), lambda i: (i, 0)),
            pl.BlockSpec((_BN, 3), lambda i: (i, 0)),
            pl.BlockSpec((_C, _Q), lambda i: (0, 0)),
            pl.BlockSpec((_Q, _K), lambda i: (0, 0)),
        ],
        out_specs=pl.BlockSpec((_BN, 128), lambda i: (i, 0)),
        out_shape=jax.ShapeDtypeStruct((_N, 128), jnp.float32),
    )(x, pos, W_q, W_dot)


def _prep_si_body(x_ref, na_ref, wsi_ref, si_ref):
    x = x_ref[...]
    na = na_ref[...]
    xa = jnp.concatenate([x * na[:, a:a + 1] for a in range(_A)], axis=1)
    si_ref[...] = jnp.dot(xa, wsi_ref[...], preferred_element_type=jnp.float32) * _SI_SCALE


def _prep_si(x, node_attr, wsi2):
    grid = (_N // _BN,)
    return pl.pallas_call(
        _prep_si_body,
        grid=grid,
        in_specs=[
            pl.BlockSpec((_BN, _C), lambda i: (i, 0)),
            pl.BlockSpec((_BN, _A), lambda i: (i, 0)),
            pl.BlockSpec((_A * _C, _O), lambda i: (0, 0)),
        ],
        out_specs=pl.BlockSpec((_BN, _O), lambda i: (i, 0)),
        out_shape=jax.ShapeDtypeStruct((_N, _O), jnp.float32),
    )(x, node_attr, wsi2)


def _pick_chunk(epw, cap=440):
  for g in range(cap, 7, -8):
    if epw % g == 0 and g % 8 == 0:
      return g
  raise ValueError(epw)


@functools.cache
def _build_sc_gather(e0, ne):
  epw = ne // _NW
  gch = _pick_chunk(epw)
  nch = epw // gch
  mesh = plsc.VectorSubcoreMesh(core_axis_name="c", subcore_axis_name="s")

  @functools.partial(
    pl.kernel,
    mesh=mesh,
    out_type=(
        jax.ShapeDtypeStruct((ne, 128), jnp.float32),
        jax.ShapeDtypeStruct((ne, 128), jnp.float32),
    ),
    scratch_types=[
        pltpu.VMEM((epw,), jnp.int32),
        pltpu.VMEM((epw,), jnp.int32),
        pltpu.VMEM((gch, 128), jnp.float32),
        pltpu.VMEM((gch, 128), jnp.float32),
        pltpu.SemaphoreType.DMA,
        pltpu.SemaphoreType.DMA,
    ],
    compiler_params=pltpu.CompilerParams(needs_layout_passes=False),
  )
  def _sc_gather(src_h, dst_h, tt, gs_out, gd_out, src_v, dst_v, sbuf, dbuf,
                 sem1, sem2):
    wid = lax.axis_index("s") * 2 + lax.axis_index("c")
    base = wid * epw
    pltpu.sync_copy(src_h.at[pl.ds(e0 + base, epw)], src_v)
    pltpu.sync_copy(dst_h.at[pl.ds(e0 + base, epw)], dst_v)

    def chunk_body(ci, carry):
        off = ci * gch
        cp1 = pltpu.async_copy(tt.at[src_v.at[pl.ds(off, gch)]], sbuf, sem1)
        cp2 = pltpu.async_copy(tt.at[dst_v.at[pl.ds(off, gch)]], dbuf, sem2)
        cp1.wait()
        pltpu.sync_copy(sbuf, gs_out.at[pl.ds(base + off, gch)])
        cp2.wait()
        pltpu.sync_copy(dbuf, gd_out.at[pl.ds(base + off, gch)])
        return carry

    lax.fori_loop(0, nch, chunk_body, 0)

  return _sc_gather


def _edge_body(gs_ref, gd_ref, eat_ref, w1e_ref, w1a_ref, w2k_ref,
               w2v_ref, out_ref):
    f32 = jnp.float32
    bf16 = jnp.bfloat16
    gs = gs_ref[...]
    gd = gd_ref[...]

    px_r = lax.broadcasted_iota(jnp.int32, (128, _C), 0)
    px_c = lax.broadcasted_iota(jnp.int32, (128, _C), 1)
    Px = (px_r == px_c).astype(f32)
    pq_r = lax.broadcasted_iota(jnp.int32, (128, _K), 0)
    pq_c = lax.broadcasted_iota(jnp.int32, (128, _K), 1)
    Pq = (pq_r == pq_c + _C).astype(f32)
    pp_r = lax.broadcasted_iota(jnp.int32, (128, 1), 0)
    Pp = ((pp_r >= _C + _K) & (pp_r < _C + _K + 3)).astype(f32)

    xs = jnp.dot(gs, Px, preferred_element_type=f32)
    qd = jnp.dot(gd, Pq, preferred_element_type=f32)
    dp = gs - gd
    l2 = jnp.dot(dp * dp, Pp, preferred_element_type=f32) + 1e-24
    elen = jnp.sqrt(l2)

    jcol = lax.broadcasted_iota(jnp.int32, (_BE, _NB), 1).astype(f32)
    vals = (jcol + 1.0) * (_MAX_R / (_NB + 1))
    diff = (elen - vals) * ((_NB + 1) / _MAX_R)
    dd = 1.0 - diff * diff
    inside = dd > 0.0
    dd_safe = jnp.where(inside, dd, 1.0)
    emb = jnp.where(inside,
                    (_EMB_C * _SQRT_NB) * jnp.exp(-2.0 / dd_safe), 0.0)

    pre = (jnp.dot(emb, w1e_ref[...], preferred_element_type=f32)
           + lax.dot_general(eat_ref[...], w1a_ref[...],
                             (((0,), (0,)), ((), ())),
                             preferred_element_type=f32))
    h = _silu(pre * _INV_S24) * _SILU_NORM
    hk = h[:, 0:128]
    hv = h[:, 128:256]
    wk2 = jnp.dot(hk, w2k_ref[...], preferred_element_type=f32) * _INV_S128
    wv2 = jnp.dot(hv.astype(bf16), w2v_ref[...].astype(bf16),
                  preferred_element_type=f32) * _INV_S128

    rk_r = lax.broadcasted_iota(jnp.int32, (_C, _C * _K), 0)
    rk_c = lax.broadcasted_iota(jnp.int32, (_C, _C * _K), 1)
    Rk = (rk_c // _K == rk_r).astype(f32)
    sk_r = lax.broadcasted_iota(jnp.int32, (_C * _K, _K), 0)
    sk_c = lax.broadcasted_iota(jnp.int32, (_C * _K, _K), 1)
    Sk = (sk_r % _K == sk_c).astype(f32)
    xs_k = jnp.dot(xs, Rk, preferred_element_type=f32)
    kraw = jnp.dot(wk2 * xs_k, Sk, preferred_element_type=f32)

    rv_r = lax.broadcasted_iota(jnp.int32, (_C, _C * _O), 0)
    rv_c = lax.broadcasted_iota(jnp.int32, (_C, _C * _O), 1)
    Rv = (rv_c // _O == rv_r).astype(bf16)
    sv_r = lax.broadcasted_iota(jnp.int32, (_C * _O, _O), 0)
    sv_c = lax.broadcasted_iota(jnp.int32, (_C * _O, _O), 1)
    Sv = (sv_r % _O == sv_c).astype(bf16)
    xs_v = jnp.dot(xs.astype(bf16), Rv, preferred_element_type=f32)
    vraw = jnp.dot((wv2 * xs_v).astype(bf16), Sv,
                   preferred_element_type=f32)

    temp = jnp.sum(qd * kraw, axis=1, keepdims=True)
    ewc = _sus(10.0 * (1.0 - elen * (1.0 / _MAX_R)))
    t2 = ewc * temp
    expv = jnp.exp(t2)
    sexp = jnp.exp(0.5 * t2)
    num = sexp * vraw * _INV_S32
    out_ref[...] = jnp.concatenate(
        [num, expv, jnp.zeros((_BE, 95), f32)], axis=1)


def _edge_pass(gs, gd, ea_t, w1e, w1a, W2_k, W2_v, blk0):
    ne = gs.shape[0]
    grid = (ne // _BE,)
    return pl.pallas_call(
        _edge_body,
        grid=grid,
        in_specs=[
            pl.BlockSpec((_BE, 128), lambda i: (i, 0)),
            pl.BlockSpec((_BE, 128), lambda i: (i, 0)),
            pl.BlockSpec((_EA, _BE), lambda i: (0, i + blk0)),
            pl.BlockSpec((_NB, 256), lambda i: (0, 0)),
            pl.BlockSpec((_EA, 256), lambda i: (0, 0)),
            pl.BlockSpec((128, _C * _K), lambda i: (0, 0)),
            pl.BlockSpec((128, _C * _O), lambda i: (0, 0)),
        ],
        out_specs=pl.BlockSpec((_BE, 128), lambda i: (i, 0)),
        out_shape=jax.ShapeDtypeStruct((ne, 128), jnp.float32),
    )(gs, gd, ea_t, w1e, w1a, W2_k, W2_v)


@functools.cache
def _build_sc_scatterN(spec):
  epws = [ne // _NW for (_, ne) in spec]
  budget = 376
  gchs = []
  for epw in epws:
    cap = max(8, (budget // len(epws)) - (budget // len(epws)) % 8)
    gchs.append(_pick_chunk(epw, cap))
  k = len(spec)
  mesh = plsc.VectorSubcoreMesh(core_axis_name="c", subcore_axis_name="s")

  scratch = []
  for i in range(k):
    scratch.append(pltpu.VMEM((gchs[i],), jnp.int32))
    scratch.append(pltpu.VMEM((gchs[i], 128), jnp.float32))
  scratch += [
      pltpu.VMEM((8, 128), jnp.float32),
      pltpu.VMEM_SHARED((_N, 128), jnp.float32),
      pltpu.SemaphoreType.DMA,
  ]

  @functools.partial(
    pl.kernel,
    mesh=mesh,
    out_type=jax.ShapeDtypeStruct((2, _N, 128), jnp.float32),
    scratch_types=scratch,
    compiler_params=pltpu.CompilerParams(needs_layout_passes=False),
  )
  def _sc_scatter(dst_h, *args):
    rows = args[0:k]
    out_h = args[k]
    bufs = args[k + 1:k + 1 + 2 * k]
    zb = args[k + 1 + 2 * k]
    table = args[k + 2 + 2 * k]
    cid = lax.axis_index("c")
    sid = lax.axis_index("s")
    wid = sid * 2 + cid

    zero16 = jnp.zeros((16,), jnp.float32)

    def zb_body(i, carry):
        r = i // 8
        c = (i % 8) * 16
        zb[r, pl.ds(c, 16)] = zero16
        return carry

    lax.fori_loop(0, 8 * 8, zb_body, 0)

    def zt_body(j, carry):
        pltpu.sync_copy(zb, table.at[pl.ds(sid * 632 + j * 8, 8)])
        return carry

    @pl.when(sid < 15)
    def _():
        lax.fori_loop(0, 79, zt_body, 0)

    @pl.when(sid == 15)
    def _():
        lax.fori_loop(0, 65, zt_body, 0)

    plsc.subcore_barrier()

    for i in range(k):
        e0, _ = spec[i]
        epw, gch = epws[i], gchs[i]
        dstc, rowsv = bufs[2 * i], bufs[2 * i + 1]
        rows_h = rows[i]

        def chunk_body(ci, carry, e0=e0, epw=epw, gch=gch, dstc=dstc,
                       rowsv=rowsv, rows_h=rows_h):
            off = wid * epw + ci * gch
            pltpu.sync_copy(dst_h.at[pl.ds(e0 + off, gch)], dstc)
            pltpu.sync_copy(rows_h.at[pl.ds(off, gch)], rowsv)
            pltpu.sync_copy(rowsv, table.at[dstc], add=True)
            return carry

        lax.fori_loop(0, epw // gch, chunk_body, 0)

    plsc.subcore_barrier()

    @pl.when(sid < 15)
    def _():
        pltpu.sync_copy(table.at[pl.ds(sid * 632, 632)],
                        out_h.at[cid, pl.ds(sid * 632, 632)])

    @pl.when(sid == 15)
    def _():
        pltpu.sync_copy(table.at[pl.ds(15 * 632, 520)],
                        out_h.at[cid, pl.ds(15 * 632, 520)])

  return _sc_scatter


def _combine_body(a0_ref, a1_ref, b0_ref, b1_ref, c0_ref, c1_ref, si_ref,
                  out_ref):
    s = ((a0_ref[0] + a1_ref[0]) + (b0_ref[0] + b1_ref[0])
         + (c0_ref[0] + c1_ref[0]))
    z = s[:, 32:33]
    zz = jnp.where(z == 0.0, 1.0, z)
    out_ref[...] = si_ref[...] + s[:, 0:_O] * lax.rsqrt(zz)


def _combine(s48a, s48b, s48c, si):
    grid = (_N // _BN,)
    return pl.pallas_call(
        _combine_body,
        grid=grid,
        in_specs=[
            pl.BlockSpec((1, _BN, 128), lambda i: (0, i, 0)),
            pl.BlockSpec((1, _BN, 128), lambda i: (1, i, 0)),
            pl.BlockSpec((1, _BN, 128), lambda i: (0, i, 0)),
            pl.BlockSpec((1, _BN, 128), lambda i: (1, i, 0)),
            pl.BlockSpec((1, _BN, 128), lambda i: (0, i, 0)),
            pl.BlockSpec((1, _BN, 128), lambda i: (1, i, 0)),
            pl.BlockSpec((_BN, _O), lambda i: (i, 0)),
        ],
        out_specs=pl.BlockSpec((_BN, _O), lambda i: (i, 0)),
        out_shape=jax.ShapeDtypeStruct((_N, _O), jnp.float32),
    )(s48a, s48a, s48b, s48b, s48c, s48c, si)


def kernel(x, pos, node_attr, edge_index, edge_attr, batch, W_q, W_si,
           W1_k, W2_k, W1_v, W2_v, W_dot):
    wsi2 = jnp.transpose(W_si, (1, 0, 2)).reshape(_A * _C, _O)
    src = edge_index[0]
    dst = edge_index[1]
    ea_t = jnp.transpose(edge_attr)
    w1kv = jnp.concatenate([W1_k, W1_v], axis=1)
    w1e = w1kv[:_NB]
    w1a = w1kv[_NB:]
    tt = _prep_table(x, pos, W_q, W_dot)
    si = _prep_si(x, node_attr, wsi2)
    sizes = (8 * _BE, 46 * _BE, 50 * _BE, 21 * _BE)
    starts = (0, sizes[0], sizes[0] + sizes[1], sizes[0] + sizes[1] + sizes[2])
    outs = []
    for e0, ne in zip(starts, sizes):
        gs_c, gd_c = _build_sc_gather(e0, ne)(src, dst, tt)
        outs.append(_edge_pass(gs_c, gd_c, ea_t, w1e, w1a, W2_k, W2_v,
                               e0 // _BE))
    s48a = _build_sc_scatterN(tuple(zip(starts[:2], sizes[:2])))(
        dst, outs[0], outs[1])
    s48b = _build_sc_scatterN(((starts[2], sizes[2]),))(dst, outs[2])
    s48c = _build_sc_scatterN(((starts[3], sizes[3]),))(dst, outs[3])
    return _combine(s48a, s48b, s48c, si)


# speedup vs baseline: 1.5709x; 1.0003x over previous
import functools
import numpy as np
import jax
import jax.numpy as jnp
from jax import lax
from jax.experimental import pallas as pl
from jax.experimental.pallas import tpu as pltpu
from jax.experimental.pallas import tpu_sc as plsc

_N = 10000
_E = 160000
_C = 32
_A = 8
_O = 32
_Q = 16
_K = 16
_NB = 8
_EA = 16
_MAX_R = 6.0
_SILU_NORM = 1.6768
_EMB_C = 1.14136 * float(np.exp(2.0))
_SQRT_NB = float(np.sqrt(_NB))
_INV_S24 = 1.0 / float(np.sqrt(_NB + _EA))
_INV_S128 = 1.0 / float(np.sqrt(128.0))
_INV_S32 = 1.0 / float(np.sqrt(_C))
_QD_SCALE = 1.0 / (_C * float(np.sqrt(_Q * _K)))
_SI_SCALE = 1.0 / float(np.sqrt(_C * _A))

_BN = 1000
_BE = 1280

_NW = 32
_EPW = _E // _NW
_NPT = _N // 16


def _sus(x):
    safe = jnp.where(x > 0.0, x, 1.0)
    return jnp.where(x > 0.0, jnp.exp(-1.0 / safe), 0.0)


def _silu(x):
    return x / (1.0 + jnp.exp(-x))


def _prep_table_body(x_ref, pos_ref, wq_ref, wdot_ref, t_ref):
    x = x_ref[...]
    wqd = jnp.dot(wq_ref[...], wdot_ref[...], preferred_element_type=jnp.float32)
    qd = jnp.dot(x, wqd, preferred_element_type=jnp.float32) * _QD_SCALE
    t_ref[...] = jnp.concatenate(
        [x, qd, pos_ref[...], jnp.zeros((_BN, 128 - _C - _K - 3), jnp.float32)],
        axis=1)


def _prep_table(x, pos, W_q, W_dot):
    grid = (_N // _BN,)
    return pl.pallas_call(
        _prep_table_body,
        grid=grid,
        in_specs=[
            pl.BlockSpec((_BN, _C), lambda i: (i, 0)),
            pl.BlockSpec((_BN, 3), lambda i: (i, 0)),
            pl.BlockSpec((_C, _Q), lambda i: (0, 0)),
            pl.BlockSpec((_Q, _K), lambda i: (0, 0)),
        ],
        out_specs=pl.BlockSpec((_BN, 128), lambda i: (i, 0)),
        out_shape=jax.ShapeDtypeStruct((_N, 128), jnp.float32),
    )(x, pos, W_q, W_dot)


def _prep_si_body(x_ref, na_ref, wsi_ref, si_ref):
    x = x_ref[...]
    na = na_ref[...]
    xa = jnp.concatenate([x * na[:, a:a + 1] for a in range(_A)], axis=1)
    si_ref[...] = jnp.dot(xa, wsi_ref[...], preferred_element_type=jnp.float32) * _SI_SCALE


def _prep_si(x, node_attr, wsi2):
    grid = (_N // _BN,)
    return pl.pallas_call(
        _prep_si_body,
        grid=grid,
        in_specs=[
            pl.BlockSpec((_BN, _C), lambda i: (i, 0)),
            pl.BlockSpec((_BN, _A), lambda i: (i, 0)),
            pl.BlockSpec((_A * _C, _O), lambda i: (0, 0)),
        ],
        out_specs=pl.BlockSpec((_BN, _O), lambda i: (i, 0)),
        out_shape=jax.ShapeDtypeStruct((_N, _O), jnp.float32),
    )(x, node_attr, wsi2)


def _pick_chunk(epw, cap=440):
  for g in range(cap, 7, -8):
    if epw % g == 0 and g % 8 == 0:
      return g
  raise ValueError(epw)


@functools.cache
def _build_sc_gather(e0, ne):
  epw = ne // _NW
  gch = _pick_chunk(epw)
  nch = epw // gch
  mesh = plsc.VectorSubcoreMesh(core_axis_name="c", subcore_axis_name="s")

  @functools.partial(
    pl.kernel,
    mesh=mesh,
    out_type=(
        jax.ShapeDtypeStruct((ne, 128), jnp.float32),
        jax.ShapeDtypeStruct((ne, 128), jnp.float32),
    ),
    scratch_types=[
        pltpu.VMEM((epw,), jnp.int32),
        pltpu.VMEM((epw,), jnp.int32),
        pltpu.VMEM((gch, 128), jnp.float32),
        pltpu.VMEM((gch, 128), jnp.float32),
        pltpu.SemaphoreType.DMA,
        pltpu.SemaphoreType.DMA,
    ],
    compiler_params=pltpu.CompilerParams(needs_layout_passes=False),
  )
  def _sc_gather(src_h, dst_h, tt, gs_out, gd_out, src_v, dst_v, sbuf, dbuf,
                 sem1, sem2):
    wid = lax.axis_index("s") * 2 + lax.axis_index("c")
    base = wid * epw
    pltpu.sync_copy(src_h.at[pl.ds(e0 + base, epw)], src_v)
    pltpu.sync_copy(dst_h.at[pl.ds(e0 + base, epw)], dst_v)

    def chunk_body(ci, carry):
        off = ci * gch
        cp1 = pltpu.async_copy(tt.at[src_v.at[pl.ds(off, gch)]], sbuf, sem1)
        cp2 = pltpu.async_copy(tt.at[dst_v.at[pl.ds(off, gch)]], dbuf, sem2)
        cp1.wait()
        pltpu.sync_copy(sbuf, gs_out.at[pl.ds(base + off, gch)])
        cp2.wait()
        pltpu.sync_copy(dbuf, gd_out.at[pl.ds(base + off, gch)])
        return carry

    lax.fori_loop(0, nch, chunk_body, 0)

  return _sc_gather


def _edge_body(gs_ref, gd_ref, eat_ref, w1e_ref, w1a_ref, w2k_ref,
               w2v_ref, out_ref):
    f32 = jnp.float32
    bf16 = jnp.bfloat16
    gs = gs_ref[...]
    gd = gd_ref[...]

    px_r = lax.broadcasted_iota(jnp.int32, (128, _C), 0)
    px_c = lax.broadcasted_iota(jnp.int32, (128, _C), 1)
    Px = (px_r == px_c).astype(f32)
    pq_r = lax.broadcasted_iota(jnp.int32, (128, _K), 0)
    pq_c = lax.broadcasted_iota(jnp.int32, (128, _K), 1)
    Pq = (pq_r == pq_c + _C).astype(f32)
    pp_r = lax.broadcasted_iota(jnp.int32, (128, 1), 0)
    Pp = ((pp_r >= _C + _K) & (pp_r < _C + _K + 3)).astype(f32)

    xs = jnp.dot(gs, Px, preferred_element_type=f32)
    qd = jnp.dot(gd, Pq, preferred_element_type=f32)
    dp = gs - gd
    l2 = jnp.dot(dp * dp, Pp, preferred_element_type=f32) + 1e-24
    elen = jnp.sqrt(l2)

    jcol = lax.broadcasted_iota(jnp.int32, (_BE, _NB), 1).astype(f32)
    vals = (jcol + 1.0) * (_MAX_R / (_NB + 1))
    diff = (elen - vals) * ((_NB + 1) / _MAX_R)
    dd = 1.0 - diff * diff
    inside = dd > 0.0
    dd_safe = jnp.where(inside, dd, 1.0)
    emb = jnp.where(inside,
                    (_EMB_C * _SQRT_NB) * jnp.exp(-2.0 / dd_safe), 0.0)

    pre = (jnp.dot(emb, w1e_ref[...], preferred_element_type=f32)
           + lax.dot_general(eat_ref[...], w1a_ref[...],
                             (((0,), (0,)), ((), ())),
                             preferred_element_type=f32))
    h = _silu(pre * _INV_S24) * _SILU_NORM
    hk = h[:, 0:128]
    hv = h[:, 128:256]
    wk2 = jnp.dot(hk, w2k_ref[...], preferred_element_type=f32) * _INV_S128
    wv2 = jnp.dot(hv.astype(bf16), w2v_ref[...].astype(bf16),
                  preferred_element_type=f32) * _INV_S128

    rk_r = lax.broadcasted_iota(jnp.int32, (_C, _C * _K), 0)
    rk_c = lax.broadcasted_iota(jnp.int32, (_C, _C * _K), 1)
    Rk = (rk_c // _K == rk_r).astype(f32)
    sk_r = lax.broadcasted_iota(jnp.int32, (_C * _K, _K), 0)
    sk_c = lax.broadcasted_iota(jnp.int32, (_C * _K, _K), 1)
    Sk = (sk_r % _K == sk_c).astype(f32)
    xs_k = jnp.dot(xs, Rk, preferred_element_type=f32)
    kraw = jnp.dot(wk2 * xs_k, Sk, preferred_element_type=f32)

    rv_r = lax.broadcasted_iota(jnp.int32, (_C, _C * _O), 0)
    rv_c = lax.broadcasted_iota(jnp.int32, (_C, _C * _O), 1)
    Rv = (rv_c // _O == rv_r).astype(bf16)
    sv_r = lax.broadcasted_iota(jnp.int32, (_C * _O, _O), 0)
    sv_c = lax.broadcasted_iota(jnp.int32, (_C * _O, _O), 1)
    Sv = (sv_r % _O == sv_c).astype(bf16)
    xs_v = jnp.dot(xs.astype(bf16), Rv, preferred_element_type=f32)
    vraw = jnp.dot((wv2 * xs_v).astype(bf16), Sv,
                   preferred_element_type=f32)

    temp = jnp.sum(qd * kraw, axis=1, keepdims=True)
    ewc = _sus(10.0 * (1.0 - elen * (1.0 / _MAX_R)))
    t2 = ewc * temp
    expv = jnp.exp(t2)
    sexp = jnp.exp(0.5 * t2)
    num = sexp * vraw * _INV_S32
    out_ref[...] = jnp.concatenate(
        [num, expv, jnp.zeros((_BE, 95), f32)], axis=1)


def _edge_pass(gs, gd, ea_t, w1e, w1a, W2_k, W2_v, blk0):
    ne = gs.shape[0]
    grid = (ne // _BE,)
    return pl.pallas_call(
        _edge_body,
        grid=grid,
        in_specs=[
            pl.BlockSpec((_BE, 128), lambda i: (i, 0)),
            pl.BlockSpec((_BE, 128), lambda i: (i, 0)),
            pl.BlockSpec((_EA, _BE), lambda i: (0, i + blk0)),
            pl.BlockSpec((_NB, 256), lambda i: (0, 0)),
            pl.BlockSpec((_EA, 256), lambda i: (0, 0)),
            pl.BlockSpec((128, _C * _K), lambda i: (0, 0)),
            pl.BlockSpec((128, _C * _O), lambda i: (0, 0)),
        ],
        out_specs=pl.BlockSpec((_BE, 128), lambda i: (i, 0)),
        out_shape=jax.ShapeDtypeStruct((ne, 128), jnp.float32),
    )(gs, gd, ea_t, w1e, w1a, W2_k, W2_v)


@functools.cache
def _build_sc_scatterN(spec):
  epws = [ne // _NW for (_, ne) in spec]
  budget = 376
  gchs = []
  for epw in epws:
    cap = max(8, (budget // len(epws)) - (budget // len(epws)) % 8)
    gchs.append(_pick_chunk(epw, cap))
  k = len(spec)
  mesh = plsc.VectorSubcoreMesh(core_axis_name="c", subcore_axis_name="s")

  scratch = []
  for i in range(k):
    scratch.append(pltpu.VMEM((gchs[i],), jnp.int32))
    scratch.append(pltpu.VMEM((gchs[i], 128), jnp.float32))
  scratch += [
      pltpu.VMEM((8, 128), jnp.float32),
      pltpu.VMEM_SHARED((_N, 128), jnp.float32),
      pltpu.SemaphoreType.DMA,
  ]

  @functools.partial(
    pl.kernel,
    mesh=mesh,
    out_type=jax.ShapeDtypeStruct((2, _N, 128), jnp.float32),
    scratch_types=scratch,
    compiler_params=pltpu.CompilerParams(needs_layout_passes=False),
  )
  def _sc_scatter(dst_h, *args):
    rows = args[0:k]
    out_h = args[k]
    bufs = args[k + 1:k + 1 + 2 * k]
    zb = args[k + 1 + 2 * k]
    table = args[k + 2 + 2 * k]
    cid = lax.axis_index("c")
    sid = lax.axis_index("s")
    wid = sid * 2 + cid

    zero16 = jnp.zeros((16,), jnp.float32)

    def zb_body(i, carry):
        r = i // 8
        c = (i % 8) * 16
        zb[r, pl.ds(c, 16)] = zero16
        return carry

    lax.fori_loop(0, 8 * 8, zb_body, 0)

    def zt_body(j, carry):
        pltpu.sync_copy(zb, table.at[pl.ds(sid * 632 + j * 8, 8)])
        return carry

    @pl.when(sid < 15)
    def _():
        lax.fori_loop(0, 79, zt_body, 0)

    @pl.when(sid == 15)
    def _():
        lax.fori_loop(0, 65, zt_body, 0)

    plsc.subcore_barrier()

    for i in range(k):
        e0, _ = spec[i]
        epw, gch = epws[i], gchs[i]
        dstc, rowsv = bufs[2 * i], bufs[2 * i + 1]
        rows_h = rows[i]

        def chunk_body(ci, carry, e0=e0, epw=epw, gch=gch, dstc=dstc,
                       rowsv=rowsv, rows_h=rows_h):
            off = wid * epw + ci * gch
            pltpu.sync_copy(dst_h.at[pl.ds(e0 + off, gch)], dstc)
            pltpu.sync_copy(rows_h.at[pl.ds(off, gch)], rowsv)
            pltpu.sync_copy(rowsv, table.at[dstc], add=True)
            return carry

        lax.fori_loop(0, epw // gch, chunk_body, 0)

    plsc.subcore_barrier()

    @pl.when(sid < 15)
    def _():
        pltpu.sync_copy(table.at[pl.ds(sid * 632, 632)],
                        out_h.at[cid, pl.ds(sid * 632, 632)])

    @pl.when(sid == 15)
    def _():
        pltpu.sync_copy(table.at[pl.ds(15 * 632, 520)],
                        out_h.at[cid, pl.ds(15 * 632, 520)])

  return _sc_scatter


def _combine_body(a0_ref, a1_ref, b0_ref, b1_ref, c0_ref, c1_ref, si_ref,
                  out_ref):
    s = ((a0_ref[0] + a1_ref[0]) + (b0_ref[0] + b1_ref[0])
         + (c0_ref[0] + c1_ref[0]))
    z = s[:, 32:33]
    zz = jnp.where(z == 0.0, 1.0, z)
    out_ref[...] = si_ref[...] + s[:, 0:_O] * lax.rsqrt(zz)


def _combine(s48a, s48b, s48c, si):
    grid = (_N // _BN,)
    return pl.pallas_call(
        _combine_body,
        grid=grid,
        in_specs=[
            pl.BlockSpec((1, _BN, 128), lambda i: (0, i, 0)),
            pl.BlockSpec((1, _BN, 128), lambda i: (1, i, 0)),
            pl.BlockSpec((1, _BN, 128), lambda i: (0, i, 0)),
            pl.BlockSpec((1, _BN, 128), lambda i: (1, i, 0)),
            pl.BlockSpec((1, _BN, 128), lambda i: (0, i, 0)),
            pl.BlockSpec((1, _BN, 128), lambda i: (1, i, 0)),
            pl.BlockSpec((_BN, _O), lambda i: (i, 0)),
        ],
        out_specs=pl.BlockSpec((_BN, _O), lambda i: (i, 0)),
        out_shape=jax.ShapeDtypeStruct((_N, _O), jnp.float32),
    )(s48a, s48a, s48b, s48b, s48c, s48c, si)


def kernel(x, pos, node_attr, edge_index, edge_attr, batch, W_q, W_si,
           W1_k, W2_k, W1_v, W2_v, W_dot):
    wsi2 = jnp.transpose(W_si, (1, 0, 2)).reshape(_A * _C, _O)
    src = edge_index[0]
    dst = edge_index[1]
    ea_t = jnp.transpose(edge_attr)
    w1kv = jnp.concatenate([W1_k, W1_v], axis=1)
    w1e = w1kv[:_NB]
    w1a = w1kv[_NB:]
    tt = _prep_table(x, pos, W_q, W_dot)
    si = _prep_si(x, node_attr, wsi2)
    sizes = (8 * _BE, 46 * _BE, 50 * _BE, 21 * _BE)
    starts = (0, sizes[0], sizes[0] + sizes[1], sizes[0] + sizes[1] + sizes[2])
    outs = []
    for e0, ne in zip(starts, sizes):
        gs_c, gd_c = _build_sc_gather(e0, ne)(src, dst, tt)
        outs.append(_edge_pass(gs_c, gd_c, ea_t, w1e, w1a, W2_k, W2_v,
                               e0 // _BE))
    s48a = _build_sc_scatterN(tuple(zip(starts[:2], sizes[:2])))(
        dst, outs[0], outs[1])
    s48b = _build_sc_scatterN(((starts[2], sizes[2]),))(dst, outs[2])
    s48c = _build_sc_scatterN(((starts[3], sizes[3]),))(dst, outs[3])
    return _combine(s48a, s48b, s48c, si)
